# Initial kernel scaffold; baseline (speedup 1.0000x reference)
import jax
import jax.numpy as jnp
from jax.experimental import pallas as pl


def kernel(x, pre_x, edge_index, edge_attr, batch, W_node, b_node, W_nn, b_nn, W_edge0, b_edge0, W_edge1, b_edge1, W_edge2, b_edge2, W1, b1, W2, b2):
    raise NotImplementedError("write your pallas kernel here")


# speedup vs baseline: 1.1813x; 1.1813x over previous
import functools

import jax
import jax.numpy as jnp
from jax import lax
from jax.experimental import pallas as pl
from jax.experimental.pallas import tpu as pltpu
from jax.experimental.pallas import tpu_sc as plsc

_N = 50000
_E = 800000
_EPAD = 819200
_EPT = _EPAD // 16
_B = 512
_NB = _EPT // _B
_RPT = 3128
_NP = 16 * _RPT
_H = 64
_HH = 32
_BN = 400
_GN = _N // _BN
_BE = 3200
_GE = _EPAD // _BE
_NG = 256
_PREC = lax.Precision.HIGHEST


def _embed_body(x_ref, p_ref, wx_ref, wp_ref, b_ref, oa_ref, ob_ref):
    h = (jnp.dot(x_ref[...], wx_ref[...], precision=_PREC,
                 preferred_element_type=jnp.float32)
         + jnp.dot(p_ref[...], wp_ref[...], precision=_PREC,
                   preferred_element_type=jnp.float32)
         + b_ref[...])
    oa_ref[...] = h[:, :_HH]
    ob_ref[...] = h[:, _HH:]


def _node_embed(x, pre_x, W_node, b_node):
    wx = W_node[:128]
    wp = W_node[128:]
    return pl.pallas_call(
        _embed_body,
        grid=(_GN,),
        in_specs=[
            pl.BlockSpec((_BN, 128), lambda i: (i, 0)),
            pl.BlockSpec((_BN, 256), lambda i: (i, 0)),
            pl.BlockSpec((128, _H), lambda i: (0, 0)),
            pl.BlockSpec((256, _H), lambda i: (0, 0)),
            pl.BlockSpec((_H,), lambda i: (0,)),
        ],
        out_specs=[
            pl.BlockSpec((_BN, _HH), lambda i: (i, 0)),
            pl.BlockSpec((_BN, _HH), lambda i: (i, 0)),
        ],
        out_shape=[
            jax.ShapeDtypeStruct((_N, _HH), jnp.float32),
            jax.ShapeDtypeStruct((_N, _HH), jnp.float32),
        ],
    )(x, pre_x, wx, wp, b_node)


def _eproj_body(a_ref, w_ref, b_ref, o_ref):
    i = pl.program_id(1)
    e = jnp.dot(a_ref[...], w_ref[0], precision=_PREC,
                preferred_element_type=jnp.float32) + b_ref[0, 0]
    o_ref[...] = jnp.where(i < _E // _BE, e, jnp.float32(-1e9))


def _edge_proj(attr_p, We, be):
    w3 = We.T.reshape(2, _HH, 16).transpose(0, 2, 1)
    b3 = be.reshape(2, 1, _HH)
    return pl.pallas_call(
        _eproj_body,
        grid=(2, _GE),
        in_specs=[
            pl.BlockSpec((_BE, 16), lambda c, i: (i, 0)),
            pl.BlockSpec((1, 16, _HH), lambda c, i: (c, 0, 0)),
            pl.BlockSpec((1, 1, _HH), lambda c, i: (c, 0, 0)),
        ],
        out_specs=pl.BlockSpec((_BE, _HH), lambda c, i: (c * _GE + i, 0)),
        out_shape=jax.ShapeDtypeStruct((2 * _EPAD, _HH), jnp.float32),
    )(attr_p, w3, b3)


def _layer_body(ha_ref, hb_ref, aa_ref, ab_ref, wa_ref, wb_ref, b_ref,
                oa_ref, ob_ref):
    za = ha_ref[...] + aa_ref[...]
    zb = hb_ref[...] + ab_ref[...]
    h = (jnp.dot(za, wa_ref[...], precision=_PREC,
                 preferred_element_type=jnp.float32)
         + jnp.dot(zb, wb_ref[...], precision=_PREC,
                   preferred_element_type=jnp.float32)
         + b_ref[...])
    h = jnp.maximum(h, 0.0)
    oa_ref[...] = h[:, :_HH]
    ob_ref[...] = h[:, _HH:]


def _layer_mlp(ha, hb, aa, ab, W_nn, b_nn):
    wa = W_nn[:_HH]
    wb = W_nn[_HH:]
    return pl.pallas_call(
        _layer_body,
        grid=(_GN,),
        in_specs=[
            pl.BlockSpec((_BN, _HH), lambda i: (i, 0)),
            pl.BlockSpec((_BN, _HH), lambda i: (i, 0)),
            pl.BlockSpec((_BN, _HH), lambda i: (i, 0)),
            pl.BlockSpec((_BN, _HH), lambda i: (i, 0)),
            pl.BlockSpec((_HH, _H), lambda i: (0, 0)),
            pl.BlockSpec((_HH, _H), lambda i: (0, 0)),
            pl.BlockSpec((_H,), lambda i: (0,)),
        ],
        out_specs=[
            pl.BlockSpec((_BN, _HH), lambda i: (i, 0)),
            pl.BlockSpec((_BN, _HH), lambda i: (i, 0)),
        ],
        out_shape=[
            jax.ShapeDtypeStruct((_N, _HH), jnp.float32),
            jax.ShapeDtypeStruct((_N, _HH), jnp.float32),
        ],
    )(ha, hb, aa, ab, wa, wb, b_nn)


def _readout_body(ha_ref, hb_ref, bt_ref, w1a_ref, w1b_ref, b1_ref,
                  w2_ref, b2_ref, o_ref):
    i = pl.program_id(0)

    @pl.when(i == 0)
    def _():
        o_ref[...] = jnp.zeros_like(o_ref)

    t = (jnp.dot(ha_ref[...], w1a_ref[...], precision=_PREC,
                 preferred_element_type=jnp.float32)
         + jnp.dot(hb_ref[...], w1b_ref[...], precision=_PREC,
                   preferred_element_type=jnp.float32)
         + b1_ref[...])
    t = jnp.maximum(t, 0.0)
    y = jnp.dot(t, w2_ref[...], precision=_PREC,
                preferred_element_type=jnp.float32) + b2_ref[...]
    bt = bt_ref[0, 0, :]
    onehot = (bt[:, None] ==
              lax.broadcasted_iota(jnp.int32, (1, _NG), 1)).astype(jnp.float32)
    o_ref[...] += lax.dot_general(onehot, y, (((0,), (0,)), ((), ())),
                                  precision=_PREC,
                                  preferred_element_type=jnp.float32)


def _readout(ha, hb, batch, W1, b1, W2, b2):
    w1a = W1[:_HH]
    w1b = W1[_HH:]
    bt3 = batch.reshape(_GN, 1, _BN)
    return pl.pallas_call(
        _readout_body,
        grid=(_GN,),
        in_specs=[
            pl.BlockSpec((_BN, _HH), lambda i: (i, 0)),
            pl.BlockSpec((_BN, _HH), lambda i: (i, 0)),
            pl.BlockSpec((1, 1, _BN), lambda i: (i, 0, 0)),
            pl.BlockSpec((_HH, 1024), lambda i: (0, 0)),
            pl.BlockSpec((_HH, 1024), lambda i: (0, 0)),
            pl.BlockSpec((1024,), lambda i: (0,)),
            pl.BlockSpec((1024, 128), lambda i: (0, 0)),
            pl.BlockSpec((128,), lambda i: (0,)),
        ],
        out_specs=pl.BlockSpec((_NG, 128), lambda i: (0, 0)),
        out_shape=jax.ShapeDtypeStruct((_NG, 128), jnp.float32),
    )(ha, hb, bt3, w1a, w1b, b1, W2, b2)


def _sc_aggr(hid2, e2, src_p, dst2, zrows):
    mesh = plsc.VectorSubcoreMesh(core_axis_name="c", subcore_axis_name="s")

    def body(hid_hbm, e_hbm, src_hbm, dst_hbm, z_hbm, out_hbm,
             srcv, dstv, ev, acc):
        c = lax.axis_index("c")
        s = lax.axis_index("s")
        pltpu.sync_copy(z_hbm, acc.at[pl.ds(s * _RPT, _RPT)])
        plsc.subcore_barrier()
        coff = c * _N
        ebase0 = s * _EPT

        def batch(i, carry):
            base = ebase0 + i * _B
            rbase = s * (_EPT // 128) + i * (_B // 128)
            pltpu.sync_copy(src_hbm.at[pl.ds(base, _B)], srcv)
            pltpu.sync_copy(dst_hbm.at[pl.ds(rbase, _B // 128)], dstv)
            for k in range(_B // 16):
                srcv[pl.ds(k * 16, 16)] = srcv[pl.ds(k * 16, 16)] + coff
            pltpu.sync_copy(e_hbm.at[pl.ds(c * _EPAD + base, _B)], ev)
            for j in range(_B // 128):
                pltpu.sync_copy(hid_hbm.at[srcv.at[pl.ds(j * 128, 128)]],
                                ev.at[pl.ds(j * 128, 128)], add=True)

            def relu_row(r, c2):
                ev[r, pl.ds(0, 16)] = jnp.maximum(ev[r, pl.ds(0, 16)], 0.0)
                ev[r, pl.ds(16, 16)] = jnp.maximum(ev[r, pl.ds(16, 16)], 0.0)
                return c2

            lax.fori_loop(0, _B, relu_row, 0, unroll=4)
            for j in range(_B // 128):
                pltpu.sync_copy(ev.at[pl.ds(j * 128, 128)],
                                acc.at[dstv.at[j]], add=True)
            return carry

        lax.fori_loop(0, _NB, batch, 0)
        plsc.subcore_barrier()
        pltpu.sync_copy(acc.at[pl.ds(s * _RPT, _RPT)],
                        out_hbm.at[pl.ds(c * _NP + s * _RPT, _RPT)])

    f = pl.kernel(
        body,
        out_type=jax.ShapeDtypeStruct((2 * _NP, _HH), jnp.float32),
        mesh=mesh,
        compiler_params=pltpu.CompilerParams(use_tc_tiling_on_sc=False),
        scratch_types=[
            pltpu.VMEM((_B,), jnp.int32),
            pltpu.VMEM((_B // 128, 128), jnp.int32),
            pltpu.VMEM((_B, _HH), jnp.float32),
            pltpu.VMEM_SHARED((_NP, _HH), jnp.float32),
        ],
    )
    return f(hid2, e2, src_p, dst2, zrows)


def kernel(x, pre_x, edge_index, edge_attr, batch,
           W_node, b_node, W_nn, b_nn,
           W_edge0, b_edge0, W_edge1, b_edge1, W_edge2, b_edge2,
           W1, b1, W2, b2):
    npad = _EPAD - _E
    src_p = jnp.concatenate([edge_index[0], jnp.zeros((npad,), jnp.int32)])
    dst_p = jnp.concatenate([edge_index[1],
                             jnp.arange(npad, dtype=jnp.int32)])
    dst2 = dst_p.reshape(_EPAD // 128, 128)
    attr_p = jnp.concatenate(
        [edge_attr, jnp.zeros((npad, 16), jnp.float32)], axis=0)
    zrows = jnp.zeros((_RPT, _HH), jnp.float32)

    ha, hb = _node_embed(x, pre_x, W_node, b_node)
    edge_lins = ((W_edge0, b_edge0), (W_edge1, b_edge1), (W_edge2, b_edge2))
    for We, be in edge_lins:
        e2 = _edge_proj(attr_p, We, be)
        hid2 = jnp.concatenate([ha, hb], axis=0)
        aggr2 = _sc_aggr(hid2, e2, src_p, dst2, zrows)
        aa = aggr2[:_N]
        ab = aggr2[_NP:_NP + _N]
        ha, hb = _layer_mlp(ha, hb, aa, ab, W_nn, b_nn)
    return _readout(ha, hb, batch, W1, b1, W2, b2)


# speedup vs baseline: 1.4560x; 1.2326x over previous
import functools

import jax
import jax.numpy as jnp
from jax import lax
from jax.experimental import pallas as pl
from jax.experimental.pallas import tpu as pltpu
from jax.experimental.pallas import tpu_sc as plsc

_N = 50000
_E = 800000
_EPAD = 819200
_EPT = _EPAD // 16
_B = 256
_NB = _EPT // _B
_RPT = 3128
_NP = 16 * _RPT
_H = 64
_HH = 32
_BN = 400
_GN = _N // _BN
_BE4 = 800
_GE4 = _EPAD // 4 // _BE4
_RE4 = _E // 4 // _BE4
_NG = 256
_PREC = lax.Precision.HIGHEST


def _embed_body(x_ref, p_ref, wx_ref, wp_ref, b_ref, oa_ref, ob_ref):
    h = (jnp.dot(x_ref[...], wx_ref[...], precision=_PREC,
                 preferred_element_type=jnp.float32)
         + jnp.dot(p_ref[...], wp_ref[...], precision=_PREC,
                   preferred_element_type=jnp.float32)
         + b_ref[...])
    oa_ref[...] = h[:, :_HH]
    ob_ref[...] = h[:, _HH:]


def _node_embed(x, pre_x, W_node, b_node):
    wx = W_node[:128]
    wp = W_node[128:]
    return pl.pallas_call(
        _embed_body,
        grid=(_GN,),
        in_specs=[
            pl.BlockSpec((_BN, 128), lambda i: (i, 0)),
            pl.BlockSpec((_BN, 256), lambda i: (i, 0)),
            pl.BlockSpec((128, _H), lambda i: (0, 0)),
            pl.BlockSpec((256, _H), lambda i: (0, 0)),
            pl.BlockSpec((_H,), lambda i: (0,)),
        ],
        out_specs=[
            pl.BlockSpec((_BN, _HH), lambda i: (i, 0)),
            pl.BlockSpec((_BN, _HH), lambda i: (i, 0)),
        ],
        out_shape=[
            jax.ShapeDtypeStruct((_N, _HH), jnp.float32),
            jax.ShapeDtypeStruct((_N, _HH), jnp.float32),
        ],
    )(x, pre_x, wx, wp, b_node)


def _eproj_body(a_ref, w_ref, b_ref, o_ref):
    i = pl.program_id(1)
    e = jnp.dot(a_ref[...], w_ref[0], precision=_PREC,
                preferred_element_type=jnp.float32) + b_ref[0, 0]
    o_ref[...] = jnp.where(i < _RE4, e, jnp.float32(-1e9))


def _edge_proj(attr4, We, be):
    eye4 = jnp.eye(4, dtype=jnp.float32)
    w4 = jnp.stack([jnp.kron(eye4, We[:, :_HH]), jnp.kron(eye4, We[:, _HH:])])
    b4 = jnp.stack([jnp.tile(be[:_HH], 4), jnp.tile(be[_HH:], 4)])
    b4 = b4.reshape(2, 1, 128)
    return pl.pallas_call(
        _eproj_body,
        grid=(2, _GE4),
        in_specs=[
            pl.BlockSpec((_BE4, 64), lambda c, i: (jnp.minimum(i, _RE4 - 1), 0)),
            pl.BlockSpec((1, 64, 128), lambda c, i: (c, 0, 0)),
            pl.BlockSpec((1, 1, 128), lambda c, i: (c, 0, 0)),
        ],
        out_specs=pl.BlockSpec((_BE4, 128), lambda c, i: (c * _GE4 + i, 0)),
        out_shape=jax.ShapeDtypeStruct((2 * _EPAD // 4, 128), jnp.float32),
    )(attr4, w4, b4)


def _layer_body(ha_ref, hb_ref, aa_ref, ab_ref, wa_ref, wb_ref, b_ref,
                oa_ref, ob_ref):
    za = ha_ref[...] + aa_ref[...]
    zb = hb_ref[...] + ab_ref[...]
    h = (jnp.dot(za, wa_ref[...], precision=_PREC,
                 preferred_element_type=jnp.float32)
         + jnp.dot(zb, wb_ref[...], precision=_PREC,
                   preferred_element_type=jnp.float32)
         + b_ref[...])
    h = jnp.maximum(h, 0.0)
    oa_ref[...] = h[:, :_HH]
    ob_ref[...] = h[:, _HH:]


def _layer_mlp(ha, hb, aa, ab, W_nn, b_nn):
    wa = W_nn[:_HH]
    wb = W_nn[_HH:]
    return pl.pallas_call(
        _layer_body,
        grid=(_GN,),
        in_specs=[
            pl.BlockSpec((_BN, _HH), lambda i: (i, 0)),
            pl.BlockSpec((_BN, _HH), lambda i: (i, 0)),
            pl.BlockSpec((_BN, _HH), lambda i: (i, 0)),
            pl.BlockSpec((_BN, _HH), lambda i: (i, 0)),
            pl.BlockSpec((_HH, _H), lambda i: (0, 0)),
            pl.BlockSpec((_HH, _H), lambda i: (0, 0)),
            pl.BlockSpec((_H,), lambda i: (0,)),
        ],
        out_specs=[
            pl.BlockSpec((_BN, _HH), lambda i: (i, 0)),
            pl.BlockSpec((_BN, _HH), lambda i: (i, 0)),
        ],
        out_shape=[
            jax.ShapeDtypeStruct((_N, _HH), jnp.float32),
            jax.ShapeDtypeStruct((_N, _HH), jnp.float32),
        ],
    )(ha, hb, aa, ab, wa, wb, b_nn)


def _readout_body(ha_ref, hb_ref, bt_ref, w1a_ref, w1b_ref, b1_ref,
                  w2_ref, b2_ref, o_ref):
    i = pl.program_id(0)

    @pl.when(i == 0)
    def _():
        o_ref[...] = jnp.zeros_like(o_ref)

    t = (jnp.dot(ha_ref[...], w1a_ref[...],
                 preferred_element_type=jnp.float32)
         + jnp.dot(hb_ref[...], w1b_ref[...],
                   preferred_element_type=jnp.float32)
         + b1_ref[...])
    t = jnp.maximum(t, 0.0)
    y = jnp.dot(t, w2_ref[...],
                preferred_element_type=jnp.float32) + b2_ref[...]
    bt = bt_ref[0, 0, :]
    onehot = (bt[:, None] ==
              lax.broadcasted_iota(jnp.int32, (1, _NG), 1)).astype(jnp.float32)
    o_ref[...] += lax.dot_general(onehot, y, (((0,), (0,)), ((), ())),
                                  precision=_PREC,
                                  preferred_element_type=jnp.float32)


def _readout(ha, hb, batch, W1, b1, W2, b2):
    w1a = W1[:_HH]
    w1b = W1[_HH:]
    bt3 = batch.reshape(_GN, 1, _BN)
    return pl.pallas_call(
        _readout_body,
        grid=(_GN,),
        in_specs=[
            pl.BlockSpec((_BN, _HH), lambda i: (i, 0)),
            pl.BlockSpec((_BN, _HH), lambda i: (i, 0)),
            pl.BlockSpec((1, 1, _BN), lambda i: (i, 0, 0)),
            pl.BlockSpec((_HH, 1024), lambda i: (0, 0)),
            pl.BlockSpec((_HH, 1024), lambda i: (0, 0)),
            pl.BlockSpec((1024,), lambda i: (0,)),
            pl.BlockSpec((1024, 128), lambda i: (0, 0)),
            pl.BlockSpec((128,), lambda i: (0,)),
        ],
        out_specs=pl.BlockSpec((_NG, 128), lambda i: (0, 0)),
        out_shape=jax.ShapeDtypeStruct((_NG, 128), jnp.float32),
    )(ha, hb, bt3, w1a, w1b, b1, W2, b2)


def _sc_aggr(hid2, e2, src_p, dst2, zrows):
    mesh = plsc.VectorSubcoreMesh(core_axis_name="c", subcore_axis_name="s")

    def body(hid_hbm, e_hbm, src_hbm, dst_hbm, z_hbm, out_hbm,
             srcv, dstv, ev, epv, acc):
        c = lax.axis_index("c")
        s = lax.axis_index("s")
        pltpu.sync_copy(z_hbm, acc.at[pl.ds(s * _RPT, _RPT)])
        plsc.subcore_barrier()
        coff = c * _N
        ebase0 = s * _EPT

        def batch(i, carry):
            base = ebase0 + i * _B
            rbase = s * (_EPT // 128) + i * (_B // 128)
            pltpu.sync_copy(src_hbm.at[pl.ds(base, _B)], srcv)
            pltpu.sync_copy(dst_hbm.at[pl.ds(rbase, _B // 128)], dstv)
            for k in range(_B // 16):
                srcv[pl.ds(k * 16, 16)] = srcv[pl.ds(k * 16, 16)] + coff
            pltpu.sync_copy(
                e_hbm.at[pl.ds(c * (_EPAD // 4) + base // 4, _B // 4)], epv)
            for j in range(_B // 128):
                pltpu.sync_copy(hid_hbm.at[srcv.at[pl.ds(j * 128, 128)]],
                                ev.at[pl.ds(j * 128, 128)])

            def fuse_row(r, c2):
                for j in range(4):
                    for kk in range(2):
                        v = (ev[r * 4 + j, pl.ds(kk * 16, 16)]
                             + epv[r, pl.ds(j * 32 + kk * 16, 16)])
                        ev[r * 4 + j, pl.ds(kk * 16, 16)] = jnp.maximum(v, 0.0)
                return c2

            lax.fori_loop(0, _B // 4, fuse_row, 0, unroll=2)
            for j in range(_B // 128):
                pltpu.sync_copy(ev.at[pl.ds(j * 128, 128)],
                                acc.at[dstv.at[j]], add=True)
            return carry

        lax.fori_loop(0, _NB, batch, 0)
        plsc.subcore_barrier()
        pltpu.sync_copy(acc.at[pl.ds(s * _RPT, _RPT)],
                        out_hbm.at[pl.ds(c * _NP + s * _RPT, _RPT)])

    f = pl.kernel(
        body,
        out_type=jax.ShapeDtypeStruct((2 * _NP, _HH), jnp.float32),
        mesh=mesh,
        compiler_params=pltpu.CompilerParams(use_tc_tiling_on_sc=False),
        scratch_types=[
            pltpu.VMEM((_B,), jnp.int32),
            pltpu.VMEM((_B // 128, 128), jnp.int32),
            pltpu.VMEM((_B, _HH), jnp.float32),
            pltpu.VMEM((_B // 4, 128), jnp.float32),
            pltpu.VMEM_SHARED((_NP, _HH), jnp.float32),
        ],
    )
    return f(hid2, e2, src_p, dst2, zrows)


def kernel(x, pre_x, edge_index, edge_attr, batch,
           W_node, b_node, W_nn, b_nn,
           W_edge0, b_edge0, W_edge1, b_edge1, W_edge2, b_edge2,
           W1, b1, W2, b2):
    npad = _EPAD - _E
    src_p = jnp.concatenate([edge_index[0], jnp.zeros((npad,), jnp.int32)])
    dst_p = jnp.concatenate([edge_index[1],
                             jnp.arange(npad, dtype=jnp.int32)])
    dst2 = dst_p.reshape(_EPAD // 128, 128)
    attr4 = edge_attr.reshape(_E // 4, 64)
    zrows = jnp.zeros((_RPT, _HH), jnp.float32)

    ha, hb = _node_embed(x, pre_x, W_node, b_node)
    edge_lins = ((W_edge0, b_edge0), (W_edge1, b_edge1), (W_edge2, b_edge2))
    for We, be in edge_lins:
        e2 = _edge_proj(attr4, We, be)
        hid2 = jnp.concatenate([ha, hb], axis=0)
        aggr2 = _sc_aggr(hid2, e2, src_p, dst2, zrows)
        aa = aggr2[:_N]
        ab = aggr2[_NP:_NP + _N]
        ha, hb = _layer_mlp(ha, hb, aa, ab, W_nn, b_nn)
    return _readout(ha, hb, batch, W1, b1, W2, b2)


# speedup vs baseline: 2.1336x; 1.4654x over previous
import functools

import jax
import jax.numpy as jnp
from jax import lax
from jax.experimental import pallas as pl
from jax.experimental.pallas import tpu as pltpu
from jax.experimental.pallas import tpu_sc as plsc

_N = 50000
_E = 800000
_EPAD = 819200
_EPT = _EPAD // 16
_B = 256
_NB = _EPT // _B
_RPT = 3128
_NP = 16 * _RPT
_H = 64
_HH = 32
_BN = 400
_GN = _N // _BN
_BE4 = 800
_GE4 = _EPAD // 4 // _BE4
_RE4 = _E // 4 // _BE4
_NG = 256
_PREC = lax.Precision.HIGHEST


def _embed_body(x_ref, p_ref, wx_ref, wp_ref, b_ref, oa_ref, ob_ref):
    h = (jnp.dot(x_ref[...], wx_ref[...], precision=_PREC,
                 preferred_element_type=jnp.float32)
         + jnp.dot(p_ref[...], wp_ref[...], precision=_PREC,
                   preferred_element_type=jnp.float32)
         + b_ref[...])
    oa_ref[...] = h[:, :_HH]
    ob_ref[...] = h[:, _HH:]


def _node_embed(x, pre_x, W_node, b_node):
    wx = W_node[:128]
    wp = W_node[128:]
    return pl.pallas_call(
        _embed_body,
        grid=(_GN,),
        in_specs=[
            pl.BlockSpec((_BN, 128), lambda i: (i, 0)),
            pl.BlockSpec((_BN, 256), lambda i: (i, 0)),
            pl.BlockSpec((128, _H), lambda i: (0, 0)),
            pl.BlockSpec((256, _H), lambda i: (0, 0)),
            pl.BlockSpec((_H,), lambda i: (0,)),
        ],
        out_specs=[
            pl.BlockSpec((_BN, _HH), lambda i: (i, 0)),
            pl.BlockSpec((_BN, _HH), lambda i: (i, 0)),
        ],
        out_shape=[
            jax.ShapeDtypeStruct((_N, _HH), jnp.float32),
            jax.ShapeDtypeStruct((_N, _HH), jnp.float32),
        ],
    )(x, pre_x, wx, wp, b_node)


def _eproj_body(a_ref, w_ref, b_ref, o_ref):
    i = pl.program_id(1)
    e = jnp.dot(a_ref[...], w_ref[0], precision=_PREC,
                preferred_element_type=jnp.float32) + b_ref[0, 0]
    o_ref[...] = jnp.where(i < _RE4, e, jnp.float32(-1e9))


def _edge_proj(attr4, We, be):
    eye4 = jnp.eye(4, dtype=jnp.float32)
    w4 = jnp.stack([jnp.kron(eye4, We[:, :_HH]), jnp.kron(eye4, We[:, _HH:])])
    b4 = jnp.stack([jnp.tile(be[:_HH], 4), jnp.tile(be[_HH:], 4)])
    b4 = b4.reshape(2, 1, 128)
    return pl.pallas_call(
        _eproj_body,
        grid=(2, _GE4),
        in_specs=[
            pl.BlockSpec((_BE4, 64), lambda c, i: (jnp.minimum(i, _RE4 - 1), 0)),
            pl.BlockSpec((1, 64, 128), lambda c, i: (c, 0, 0)),
            pl.BlockSpec((1, 1, 128), lambda c, i: (c, 0, 0)),
        ],
        out_specs=pl.BlockSpec((_BE4, 128), lambda c, i: (c * _GE4 + i, 0)),
        out_shape=jax.ShapeDtypeStruct((2 * _EPAD // 4, 128), jnp.float32),
    )(attr4, w4, b4)


def _layer_body(ha_ref, hb_ref, aa_ref, ab_ref, wa_ref, wb_ref, b_ref,
                oa_ref, ob_ref):
    za = ha_ref[...] + aa_ref[...]
    zb = hb_ref[...] + ab_ref[...]
    h = (jnp.dot(za, wa_ref[...], precision=_PREC,
                 preferred_element_type=jnp.float32)
         + jnp.dot(zb, wb_ref[...], precision=_PREC,
                   preferred_element_type=jnp.float32)
         + b_ref[...])
    h = jnp.maximum(h, 0.0)
    oa_ref[...] = h[:, :_HH]
    ob_ref[...] = h[:, _HH:]


def _layer_mlp(ha, hb, aa, ab, W_nn, b_nn):
    wa = W_nn[:_HH]
    wb = W_nn[_HH:]
    return pl.pallas_call(
        _layer_body,
        grid=(_GN,),
        in_specs=[
            pl.BlockSpec((_BN, _HH), lambda i: (i, 0)),
            pl.BlockSpec((_BN, _HH), lambda i: (i, 0)),
            pl.BlockSpec((_BN, _HH), lambda i: (i, 0)),
            pl.BlockSpec((_BN, _HH), lambda i: (i, 0)),
            pl.BlockSpec((_HH, _H), lambda i: (0, 0)),
            pl.BlockSpec((_HH, _H), lambda i: (0, 0)),
            pl.BlockSpec((_H,), lambda i: (0,)),
        ],
        out_specs=[
            pl.BlockSpec((_BN, _HH), lambda i: (i, 0)),
            pl.BlockSpec((_BN, _HH), lambda i: (i, 0)),
        ],
        out_shape=[
            jax.ShapeDtypeStruct((_N, _HH), jnp.float32),
            jax.ShapeDtypeStruct((_N, _HH), jnp.float32),
        ],
    )(ha, hb, aa, ab, wa, wb, b_nn)


def _readout_body(ha_ref, hb_ref, bt_ref, w1a_ref, w1b_ref, b1_ref,
                  w2_ref, b2_ref, o_ref):
    i = pl.program_id(0)

    @pl.when(i == 0)
    def _():
        o_ref[...] = jnp.zeros_like(o_ref)

    t = (jnp.dot(ha_ref[...], w1a_ref[...],
                 preferred_element_type=jnp.float32)
         + jnp.dot(hb_ref[...], w1b_ref[...],
                   preferred_element_type=jnp.float32)
         + b1_ref[...])
    t = jnp.maximum(t, 0.0)
    y = jnp.dot(t, w2_ref[...],
                preferred_element_type=jnp.float32) + b2_ref[...]
    bt = bt_ref[0, 0, :]
    onehot = (bt[:, None] ==
              lax.broadcasted_iota(jnp.int32, (1, _NG), 1)).astype(jnp.float32)
    o_ref[...] += lax.dot_general(onehot, y, (((0,), (0,)), ((), ())),
                                  precision=_PREC,
                                  preferred_element_type=jnp.float32)


def _readout(ha, hb, batch, W1, b1, W2, b2):
    w1a = W1[:_HH]
    w1b = W1[_HH:]
    bt3 = batch.reshape(_GN, 1, _BN)
    return pl.pallas_call(
        _readout_body,
        grid=(_GN,),
        in_specs=[
            pl.BlockSpec((_BN, _HH), lambda i: (i, 0)),
            pl.BlockSpec((_BN, _HH), lambda i: (i, 0)),
            pl.BlockSpec((1, 1, _BN), lambda i: (i, 0, 0)),
            pl.BlockSpec((_HH, 1024), lambda i: (0, 0)),
            pl.BlockSpec((_HH, 1024), lambda i: (0, 0)),
            pl.BlockSpec((1024,), lambda i: (0,)),
            pl.BlockSpec((1024, 128), lambda i: (0, 0)),
            pl.BlockSpec((128,), lambda i: (0,)),
        ],
        out_specs=pl.BlockSpec((_NG, 128), lambda i: (0, 0)),
        out_shape=jax.ShapeDtypeStruct((_NG, 128), jnp.float32),
    )(ha, hb, bt3, w1a, w1b, b1, W2, b2)


def _sc_aggr(hid2, e2, src_p, dst2, zrows):
    mesh = plsc.VectorSubcoreMesh(core_axis_name="c", subcore_axis_name="s")

    def body(hid_hbm, e_hbm, src_hbm, dst_hbm, z_hbm, out_hbm,
             sva, dva, eva, svb, dvb, evb, sema, semb, acc):
        c = lax.axis_index("c")
        s = lax.axis_index("s")
        pltpu.sync_copy(z_hbm, acc.at[pl.ds(s * _RPT, _RPT)])
        plsc.subcore_barrier()
        coff = c * _N
        ebase0 = s * _EPT

        def slices(i):
            base = ebase0 + i * _B
            rbase = s * (_EPT // 128) + i * (_B // 128)
            return (src_hbm.at[pl.ds(base, _B)],
                    dst_hbm.at[pl.ds(rbase, _B // 128)],
                    e_hbm.at[pl.ds(c * _EPAD + base, _B)])

        def issue(i, sv, dv, evr, sem):
            ss, ds_, es = slices(i)
            pltpu.async_copy(ss, sv, sem)
            pltpu.async_copy(ds_, dv, sem)
            pltpu.async_copy(es, evr, sem)

        def process(i, sv, dv, evr, sem):
            ss, ds_, es = slices(i)
            pltpu.make_async_copy(ss, sv, sem).wait()
            pltpu.make_async_copy(ds_, dv, sem).wait()
            pltpu.make_async_copy(es, evr, sem).wait()
            for k in range(_B // 16):
                sv[pl.ds(k * 16, 16)] = sv[pl.ds(k * 16, 16)] + coff
            for j in range(_B // 128):
                pltpu.sync_copy(hid_hbm.at[sv.at[pl.ds(j * 128, 128)]],
                                evr.at[pl.ds(j * 128, 128)], add=True)

            def relu_row(r, c2):
                evr[r, pl.ds(0, 16)] = jnp.maximum(evr[r, pl.ds(0, 16)], 0.0)
                evr[r, pl.ds(16, 16)] = jnp.maximum(evr[r, pl.ds(16, 16)], 0.0)
                return c2

            lax.fori_loop(0, _B, relu_row, 0, unroll=4)
            for j in range(_B // 128):
                pltpu.sync_copy(evr.at[pl.ds(j * 128, 128)],
                                acc.at[dv.at[j]], add=True)

        issue(0, sva, dva, eva, sema)

        def pair(g, carry):
            i0 = g * 2
            issue(i0 + 1, svb, dvb, evb, semb)
            process(i0, sva, dva, eva, sema)

            @pl.when(g + 1 < _NB // 2)
            def _():
                issue(i0 + 2, sva, dva, eva, sema)

            process(i0 + 1, svb, dvb, evb, semb)
            return carry

        lax.fori_loop(0, _NB // 2, pair, 0)
        plsc.subcore_barrier()
        pltpu.sync_copy(acc.at[pl.ds(s * _RPT, _RPT)],
                        out_hbm.at[pl.ds(c * _NP + s * _RPT, _RPT)])

    f = pl.kernel(
        body,
        out_type=jax.ShapeDtypeStruct((2 * _NP, _HH), jnp.float32),
        mesh=mesh,
        compiler_params=pltpu.CompilerParams(use_tc_tiling_on_sc=False),
        scratch_types=[
            pltpu.VMEM((_B,), jnp.int32),
            pltpu.VMEM((_B // 128, 128), jnp.int32),
            pltpu.VMEM((_B, _HH), jnp.float32),
            pltpu.VMEM((_B,), jnp.int32),
            pltpu.VMEM((_B // 128, 128), jnp.int32),
            pltpu.VMEM((_B, _HH), jnp.float32),
            pltpu.SemaphoreType.DMA,
            pltpu.SemaphoreType.DMA,
            pltpu.VMEM_SHARED((_NP, _HH), jnp.float32),
        ],
    )
    return f(hid2, e2.reshape(2 * _EPAD, _HH), src_p, dst2, zrows)


def kernel(x, pre_x, edge_index, edge_attr, batch,
           W_node, b_node, W_nn, b_nn,
           W_edge0, b_edge0, W_edge1, b_edge1, W_edge2, b_edge2,
           W1, b1, W2, b2):
    npad = _EPAD - _E
    src_p = jnp.concatenate([edge_index[0], jnp.zeros((npad,), jnp.int32)])
    dst_p = jnp.concatenate([edge_index[1],
                             jnp.arange(npad, dtype=jnp.int32)])
    dst2 = dst_p.reshape(_EPAD // 128, 128)
    attr4 = edge_attr.reshape(_E // 4, 64)
    zrows = jnp.zeros((_RPT, _HH), jnp.float32)

    ha, hb = _node_embed(x, pre_x, W_node, b_node)
    edge_lins = ((W_edge0, b_edge0), (W_edge1, b_edge1), (W_edge2, b_edge2))
    for We, be in edge_lins:
        e2 = _edge_proj(attr4, We, be)
        hid2 = jnp.concatenate([ha, hb], axis=0)
        aggr2 = _sc_aggr(hid2, e2, src_p, dst2, zrows)
        aa = aggr2[:_N]
        ab = aggr2[_NP:_NP + _N]
        ha, hb = _layer_mlp(ha, hb, aa, ab, W_nn, b_nn)
    return _readout(ha, hb, batch, W1, b1, W2, b2)


# speedup vs baseline: 2.2237x; 1.0422x over previous
import functools

import jax
import jax.numpy as jnp
from jax import lax
from jax.experimental import pallas as pl
from jax.experimental.pallas import tpu as pltpu
from jax.experimental.pallas import tpu_sc as plsc

_N = 50000
_E = 800000
_EPAD = 819200
_EPT = _EPAD // 16
_B = 256
_NB = _EPT // _B
_RPT = 3128
_NP = 16 * _RPT
_H = 64
_HH = 32
_BN = 400
_GN = _N // _BN
_BE4 = 800
_GE4 = _EPAD // 4 // _BE4
_RE4 = _E // 4 // _BE4
_NG = 256
_PREC = lax.Precision.HIGHEST


def _embed_body(x_ref, p_ref, wx_ref, wp_ref, b_ref, oa_ref, ob_ref):
    h = (jnp.dot(x_ref[...], wx_ref[...], precision=_PREC,
                 preferred_element_type=jnp.float32)
         + jnp.dot(p_ref[...], wp_ref[...], precision=_PREC,
                   preferred_element_type=jnp.float32)
         + b_ref[...])
    oa_ref[...] = h[:, :_HH]
    ob_ref[...] = h[:, _HH:]


def _node_embed(x, pre_x, W_node, b_node):
    wx = W_node[:128]
    wp = W_node[128:]
    return pl.pallas_call(
        _embed_body,
        grid=(_GN,),
        in_specs=[
            pl.BlockSpec((_BN, 128), lambda i: (i, 0)),
            pl.BlockSpec((_BN, 256), lambda i: (i, 0)),
            pl.BlockSpec((128, _H), lambda i: (0, 0)),
            pl.BlockSpec((256, _H), lambda i: (0, 0)),
            pl.BlockSpec((_H,), lambda i: (0,)),
        ],
        out_specs=[
            pl.BlockSpec((_BN, _HH), lambda i: (i, 0)),
            pl.BlockSpec((_BN, _HH), lambda i: (i, 0)),
        ],
        out_shape=[
            jax.ShapeDtypeStruct((_N, _HH), jnp.float32),
            jax.ShapeDtypeStruct((_N, _HH), jnp.float32),
        ],
    )(x, pre_x, wx, wp, b_node)


def _eproj_body(a_ref, w_ref, b_ref, o_ref):
    i = pl.program_id(1)
    e = jnp.dot(a_ref[...], w_ref[0],
                preferred_element_type=jnp.float32) + b_ref[0, 0]
    o_ref[...] = jnp.where(i < _RE4, e, jnp.float32(-1e9))


def _edge_proj(attr4, We, be):
    eye4 = jnp.eye(4, dtype=jnp.float32)
    w4 = jnp.stack([jnp.kron(eye4, We[:, :_HH]), jnp.kron(eye4, We[:, _HH:])])
    b4 = jnp.stack([jnp.tile(be[:_HH], 4), jnp.tile(be[_HH:], 4)])
    b4 = b4.reshape(2, 1, 128)
    return pl.pallas_call(
        _eproj_body,
        grid=(2, _GE4),
        in_specs=[
            pl.BlockSpec((_BE4, 64), lambda c, i: (jnp.minimum(i, _RE4 - 1), 0)),
            pl.BlockSpec((1, 64, 128), lambda c, i: (c, 0, 0)),
            pl.BlockSpec((1, 1, 128), lambda c, i: (c, 0, 0)),
        ],
        out_specs=pl.BlockSpec((_BE4, 128), lambda c, i: (c * _GE4 + i, 0)),
        out_shape=jax.ShapeDtypeStruct((2 * _EPAD // 4, 128), jnp.float32),
    )(attr4, w4, b4)


def _layer_body(ha_ref, hb_ref, aa_ref, ab_ref, wa_ref, wb_ref, b_ref,
                oa_ref, ob_ref):
    za = ha_ref[...] + aa_ref[...]
    zb = hb_ref[...] + ab_ref[...]
    h = (jnp.dot(za, wa_ref[...], precision=_PREC,
                 preferred_element_type=jnp.float32)
         + jnp.dot(zb, wb_ref[...], precision=_PREC,
                   preferred_element_type=jnp.float32)
         + b_ref[...])
    h = jnp.maximum(h, 0.0)
    oa_ref[...] = h[:, :_HH]
    ob_ref[...] = h[:, _HH:]


def _layer_mlp(ha, hb, aa, ab, W_nn, b_nn):
    wa = W_nn[:_HH]
    wb = W_nn[_HH:]
    return pl.pallas_call(
        _layer_body,
        grid=(_GN,),
        in_specs=[
            pl.BlockSpec((_BN, _HH), lambda i: (i, 0)),
            pl.BlockSpec((_BN, _HH), lambda i: (i, 0)),
            pl.BlockSpec((_BN, _HH), lambda i: (i, 0)),
            pl.BlockSpec((_BN, _HH), lambda i: (i, 0)),
            pl.BlockSpec((_HH, _H), lambda i: (0, 0)),
            pl.BlockSpec((_HH, _H), lambda i: (0, 0)),
            pl.BlockSpec((_H,), lambda i: (0,)),
        ],
        out_specs=[
            pl.BlockSpec((_BN, _HH), lambda i: (i, 0)),
            pl.BlockSpec((_BN, _HH), lambda i: (i, 0)),
        ],
        out_shape=[
            jax.ShapeDtypeStruct((_N, _HH), jnp.float32),
            jax.ShapeDtypeStruct((_N, _HH), jnp.float32),
        ],
    )(ha, hb, aa, ab, wa, wb, b_nn)


def _readout_body(ha_ref, hb_ref, bt_ref, w1a_ref, w1b_ref, b1_ref,
                  w2_ref, b2_ref, o_ref):
    i = pl.program_id(0)

    @pl.when(i == 0)
    def _():
        o_ref[...] = jnp.zeros_like(o_ref)

    t = (jnp.dot(ha_ref[...], w1a_ref[...],
                 preferred_element_type=jnp.float32)
         + jnp.dot(hb_ref[...], w1b_ref[...],
                   preferred_element_type=jnp.float32)
         + b1_ref[...])
    t = jnp.maximum(t, 0.0)
    y = jnp.dot(t, w2_ref[...],
                preferred_element_type=jnp.float32) + b2_ref[...]
    bt = bt_ref[0, 0, :]
    onehot = (bt[:, None] ==
              lax.broadcasted_iota(jnp.int32, (1, _NG), 1)).astype(jnp.float32)
    o_ref[...] += lax.dot_general(onehot, y, (((0,), (0,)), ((), ())),
                                  precision=_PREC,
                                  preferred_element_type=jnp.float32)


def _readout(ha, hb, batch, W1, b1, W2, b2):
    w1a = W1[:_HH]
    w1b = W1[_HH:]
    bt3 = batch.reshape(_GN, 1, _BN)
    return pl.pallas_call(
        _readout_body,
        grid=(_GN,),
        in_specs=[
            pl.BlockSpec((_BN, _HH), lambda i: (i, 0)),
            pl.BlockSpec((_BN, _HH), lambda i: (i, 0)),
            pl.BlockSpec((1, 1, _BN), lambda i: (i, 0, 0)),
            pl.BlockSpec((_HH, 1024), lambda i: (0, 0)),
            pl.BlockSpec((_HH, 1024), lambda i: (0, 0)),
            pl.BlockSpec((1024,), lambda i: (0,)),
            pl.BlockSpec((1024, 128), lambda i: (0, 0)),
            pl.BlockSpec((128,), lambda i: (0,)),
        ],
        out_specs=pl.BlockSpec((_NG, 128), lambda i: (0, 0)),
        out_shape=jax.ShapeDtypeStruct((_NG, 128), jnp.float32),
    )(ha, hb, bt3, w1a, w1b, b1, W2, b2)


def _sc_aggr(hid2, e2, src_p, dst2, zrows):
    mesh = plsc.VectorSubcoreMesh(core_axis_name="c", subcore_axis_name="s")

    def body(hid_hbm, e_hbm, src_hbm, dst_hbm, z_hbm, out_hbm,
             sva, dva, eva, svb, dvb, evb, sema, semb, acc):
        c = lax.axis_index("c")
        s = lax.axis_index("s")
        pltpu.sync_copy(z_hbm, acc.at[pl.ds(s * _RPT, _RPT)])
        plsc.subcore_barrier()
        coff = c * _N
        ebase0 = s * _EPT

        def slices(i):
            base = ebase0 + i * _B
            rbase = s * (_EPT // 128) + i * (_B // 128)
            return (src_hbm.at[pl.ds(base, _B)],
                    dst_hbm.at[pl.ds(rbase, _B // 128)],
                    e_hbm.at[pl.ds(c * _EPAD + base, _B)])

        def issue(i, sv, dv, evr, sem):
            ss, ds_, es = slices(i)
            pltpu.async_copy(ss, sv, sem)
            pltpu.async_copy(ds_, dv, sem)
            pltpu.async_copy(es, evr, sem)

        def process(i, sv, dv, evr, sem):
            ss, ds_, es = slices(i)
            pltpu.make_async_copy(ss, sv, sem).wait()
            pltpu.make_async_copy(ds_, dv, sem).wait()
            pltpu.make_async_copy(es, evr, sem).wait()
            for k in range(_B // 16):
                sv[pl.ds(k * 16, 16)] = sv[pl.ds(k * 16, 16)] + coff
            for j in range(_B // 128):
                pltpu.sync_copy(hid_hbm.at[sv.at[pl.ds(j * 128, 128)]],
                                evr.at[pl.ds(j * 128, 128)], add=True)

            def relu_row(r, c2):
                evr[r, pl.ds(0, 16)] = jnp.maximum(evr[r, pl.ds(0, 16)], 0.0)
                evr[r, pl.ds(16, 16)] = jnp.maximum(evr[r, pl.ds(16, 16)], 0.0)
                return c2

            lax.fori_loop(0, _B, relu_row, 0, unroll=4)
            for j in range(_B // 128):
                pltpu.sync_copy(evr.at[pl.ds(j * 128, 128)],
                                acc.at[dv.at[j]], add=True)

        issue(0, sva, dva, eva, sema)

        def pair(g, carry):
            i0 = g * 2
            issue(i0 + 1, svb, dvb, evb, semb)
            process(i0, sva, dva, eva, sema)

            @pl.when(g + 1 < _NB // 2)
            def _():
                issue(i0 + 2, sva, dva, eva, sema)

            process(i0 + 1, svb, dvb, evb, semb)
            return carry

        lax.fori_loop(0, _NB // 2, pair, 0)
        plsc.subcore_barrier()
        pltpu.sync_copy(acc.at[pl.ds(s * _RPT, _RPT)],
                        out_hbm.at[pl.ds(c * _NP + s * _RPT, _RPT)])

    f = pl.kernel(
        body,
        out_type=jax.ShapeDtypeStruct((2 * _NP, _HH), jnp.float32),
        mesh=mesh,
        compiler_params=pltpu.CompilerParams(use_tc_tiling_on_sc=False),
        scratch_types=[
            pltpu.VMEM((_B,), jnp.int32),
            pltpu.VMEM((_B // 128, 128), jnp.int32),
            pltpu.VMEM((_B, _HH), jnp.float32),
            pltpu.VMEM((_B,), jnp.int32),
            pltpu.VMEM((_B // 128, 128), jnp.int32),
            pltpu.VMEM((_B, _HH), jnp.float32),
            pltpu.SemaphoreType.DMA,
            pltpu.SemaphoreType.DMA,
            pltpu.VMEM_SHARED((_NP, _HH), jnp.float32),
        ],
    )
    return f(hid2, e2.reshape(2 * _EPAD, _HH), src_p, dst2, zrows)


def kernel(x, pre_x, edge_index, edge_attr, batch,
           W_node, b_node, W_nn, b_nn,
           W_edge0, b_edge0, W_edge1, b_edge1, W_edge2, b_edge2,
           W1, b1, W2, b2):
    npad = _EPAD - _E
    src_p = jnp.concatenate([edge_index[0], jnp.zeros((npad,), jnp.int32)])
    dst_p = jnp.concatenate([edge_index[1],
                             jnp.arange(npad, dtype=jnp.int32)])
    dst2 = dst_p.reshape(_EPAD // 128, 128)
    attr4 = edge_attr.reshape(_E // 4, 64)
    zrows = jnp.zeros((_RPT, _HH), jnp.float32)

    ha, hb = _node_embed(x, pre_x, W_node, b_node)
    edge_lins = ((W_edge0, b_edge0), (W_edge1, b_edge1), (W_edge2, b_edge2))
    e2s = [_edge_proj(attr4, We, be) for We, be in edge_lins]
    for e2 in e2s:
        hid2 = jnp.concatenate([ha, hb], axis=0)
        aggr2 = _sc_aggr(hid2, e2, src_p, dst2, zrows)
        aa = aggr2[:_N]
        ab = aggr2[_NP:_NP + _N]
        ha, hb = _layer_mlp(ha, hb, aa, ab, W_nn, b_nn)
    return _readout(ha, hb, batch, W1, b1, W2, b2)


# speedup vs baseline: 2.7631x; 1.2426x over previous
import functools

import jax
import jax.numpy as jnp
from jax import lax
from jax.experimental import pallas as pl
from jax.experimental.pallas import tpu as pltpu
from jax.experimental.pallas import tpu_sc as plsc

_N = 50000
_E = 800000
_EPAD = 819200
_EPT = _EPAD // 16
_B = 256
_NB = _EPT // _B
_RPT = 3128
_NP = 16 * _RPT
_H = 64
_HH = 32
_BN = 400
_GN = _N // _BN
_BE4 = 800
_GE4 = _EPAD // 4 // _BE4
_RE4 = _E // 4 // _BE4
_NG = 256
_BNR = 2000
_PREC = lax.Precision.HIGHEST


def _embed_body(x_ref, p_ref, wx_ref, wp_ref, b_ref, oa_ref, ob_ref):
    h = (jnp.dot(x_ref[...], wx_ref[...], precision=_PREC,
                 preferred_element_type=jnp.float32)
         + jnp.dot(p_ref[...], wp_ref[...], precision=_PREC,
                   preferred_element_type=jnp.float32)
         + b_ref[...])
    oa_ref[...] = h[:, :_HH]
    ob_ref[...] = h[:, _HH:]


def _node_embed(x, pre_x, W_node, b_node):
    wx = W_node[:128]
    wp = W_node[128:]
    return pl.pallas_call(
        _embed_body,
        grid=(_GN,),
        in_specs=[
            pl.BlockSpec((_BN, 128), lambda i: (i, 0)),
            pl.BlockSpec((_BN, 256), lambda i: (i, 0)),
            pl.BlockSpec((128, _H), lambda i: (0, 0)),
            pl.BlockSpec((256, _H), lambda i: (0, 0)),
            pl.BlockSpec((_H,), lambda i: (0,)),
        ],
        out_specs=[
            pl.BlockSpec((_BN, _HH), lambda i: (i, 0)),
            pl.BlockSpec((_BN, _HH), lambda i: (i, 0)),
        ],
        out_shape=[
            jax.ShapeDtypeStruct((_N, _HH), jnp.float32),
            jax.ShapeDtypeStruct((_N, _HH), jnp.float32),
        ],
    )(x, pre_x, wx, wp, b_node)


def _eproj_body(a_ref, w_ref, b_ref, o_ref):
    i = pl.program_id(1)
    e = jnp.dot(a_ref[...], w_ref[0],
                preferred_element_type=jnp.float32) + b_ref[0, 0]
    o_ref[...] = jnp.where(i < _RE4, e, jnp.float32(-1e9))


def _edge_proj(attr4, We, be):
    eye4 = jnp.eye(4, dtype=jnp.float32)
    w4 = jnp.stack([jnp.kron(eye4, We[:, :_HH]), jnp.kron(eye4, We[:, _HH:])])
    b4 = jnp.stack([jnp.tile(be[:_HH], 4), jnp.tile(be[_HH:], 4)])
    b4 = b4.reshape(2, 1, 128)
    return pl.pallas_call(
        _eproj_body,
        grid=(2, _GE4),
        in_specs=[
            pl.BlockSpec((_BE4, 64), lambda c, i: (jnp.minimum(i, _RE4 - 1), 0)),
            pl.BlockSpec((1, 64, 128), lambda c, i: (c, 0, 0)),
            pl.BlockSpec((1, 1, 128), lambda c, i: (c, 0, 0)),
        ],
        out_specs=pl.BlockSpec((_BE4, 128), lambda c, i: (c * _GE4 + i, 0)),
        out_shape=jax.ShapeDtypeStruct((2 * _EPAD // 4, 128), jnp.float32),
    )(attr4, w4, b4)


def _layer_body(ha_ref, hb_ref, aa_ref, ab_ref, wa_ref, wb_ref, b_ref,
                oa_ref, ob_ref):
    za = ha_ref[...] + aa_ref[...]
    zb = hb_ref[...] + ab_ref[...]
    h = (jnp.dot(za, wa_ref[...], precision=_PREC,
                 preferred_element_type=jnp.float32)
         + jnp.dot(zb, wb_ref[...], precision=_PREC,
                   preferred_element_type=jnp.float32)
         + b_ref[...])
    h = jnp.maximum(h, 0.0)
    oa_ref[...] = h[:, :_HH]
    ob_ref[...] = h[:, _HH:]


def _layer_mlp(ha, hb, aa, ab, W_nn, b_nn):
    wa = W_nn[:_HH]
    wb = W_nn[_HH:]
    return pl.pallas_call(
        _layer_body,
        grid=(_GN,),
        in_specs=[
            pl.BlockSpec((_BN, _HH), lambda i: (i, 0)),
            pl.BlockSpec((_BN, _HH), lambda i: (i, 0)),
            pl.BlockSpec((_BN, _HH), lambda i: (i, 0)),
            pl.BlockSpec((_BN, _HH), lambda i: (i, 0)),
            pl.BlockSpec((_HH, _H), lambda i: (0, 0)),
            pl.BlockSpec((_HH, _H), lambda i: (0, 0)),
            pl.BlockSpec((_H,), lambda i: (0,)),
        ],
        out_specs=[
            pl.BlockSpec((_BN, _HH), lambda i: (i, 0)),
            pl.BlockSpec((_BN, _HH), lambda i: (i, 0)),
        ],
        out_shape=[
            jax.ShapeDtypeStruct((_N, _HH), jnp.float32),
            jax.ShapeDtypeStruct((_N, _HH), jnp.float32),
        ],
    )(ha, hb, aa, ab, wa, wb, b_nn)


def _readout_body(ha_ref, hb_ref, bt_ref, w1a_ref, w1b_ref, b1_ref,
                  w2_ref, b2_ref, o_ref):
    i = pl.program_id(0)

    @pl.when(i == 0)
    def _():
        o_ref[...] = jnp.zeros_like(o_ref)

    t = (jnp.dot(ha_ref[...], w1a_ref[...],
                 preferred_element_type=jnp.float32)
         + jnp.dot(hb_ref[...], w1b_ref[...],
                   preferred_element_type=jnp.float32)
         + b1_ref[...])
    t = jnp.maximum(t, 0.0)
    y = jnp.dot(t, w2_ref[...],
                preferred_element_type=jnp.float32) + b2_ref[...]
    bt = bt_ref[0, 0, :]
    onehot = (bt[:, None] ==
              lax.broadcasted_iota(jnp.int32, (1, _NG), 1)).astype(jnp.float32)
    o_ref[...] += lax.dot_general(onehot, y, (((0,), (0,)), ((), ())),
                                  precision=_PREC,
                                  preferred_element_type=jnp.float32)


def _readout(ha, hb, batch, W1, b1, W2, b2):
    w1a = W1[:_HH]
    w1b = W1[_HH:]
    bt3 = batch.reshape(_N // _BNR, 1, _BNR)
    return pl.pallas_call(
        _readout_body,
        grid=(_N // _BNR,),
        in_specs=[
            pl.BlockSpec((_BNR, _HH), lambda i: (i, 0)),
            pl.BlockSpec((_BNR, _HH), lambda i: (i, 0)),
            pl.BlockSpec((1, 1, _BNR), lambda i: (i, 0, 0)),
            pl.BlockSpec((_HH, 1024), lambda i: (0, 0)),
            pl.BlockSpec((_HH, 1024), lambda i: (0, 0)),
            pl.BlockSpec((1024,), lambda i: (0,)),
            pl.BlockSpec((1024, 128), lambda i: (0, 0)),
            pl.BlockSpec((128,), lambda i: (0,)),
        ],
        out_specs=pl.BlockSpec((_NG, 128), lambda i: (0, 0)),
        out_shape=jax.ShapeDtypeStruct((_NG, 128), jnp.float32),
    )(ha, hb, bt3, w1a, w1b, b1, W2, b2)


def _sc_aggr(ha, hb, e2, src_p, dst2, zrows):
    mesh = plsc.VectorSubcoreMesh(core_axis_name="c", subcore_axis_name="s")

    def body(ha_hbm, hb_hbm, e_hbm, src_hbm, dst_hbm, z_hbm,
             outa_hbm, outb_hbm,
             sva, dva, eva, svb, dvb, evb, sema, semb, acc):
        c = lax.axis_index("c")
        s = lax.axis_index("s")
        pltpu.sync_copy(z_hbm, acc.at[pl.ds(s * _RPT, _RPT)])
        plsc.subcore_barrier()
        ebase0 = s * _EPT

        def slices(i):
            base = ebase0 + i * _B
            rbase = s * (_EPT // 128) + i * (_B // 128)
            return (src_hbm.at[pl.ds(base, _B)],
                    dst_hbm.at[pl.ds(rbase, _B // 128)],
                    e_hbm.at[pl.ds(c * _EPAD + base, _B)])

        def issue(i, sv, dv, evr, sem):
            ss, ds_, es = slices(i)
            pltpu.async_copy(ss, sv, sem)
            pltpu.async_copy(ds_, dv, sem)
            pltpu.async_copy(es, evr, sem)

        def process(i, sv, dv, evr, sem):
            ss, ds_, es = slices(i)
            pltpu.make_async_copy(ss, sv, sem).wait()
            pltpu.make_async_copy(ds_, dv, sem).wait()
            pltpu.make_async_copy(es, evr, sem).wait()

            @pl.when(c == 0)
            def _():
                for j in range(_B // 128):
                    pltpu.sync_copy(ha_hbm.at[sv.at[pl.ds(j * 128, 128)]],
                                    evr.at[pl.ds(j * 128, 128)], add=True)

            @pl.when(c == 1)
            def _():
                for j in range(_B // 128):
                    pltpu.sync_copy(hb_hbm.at[sv.at[pl.ds(j * 128, 128)]],
                                    evr.at[pl.ds(j * 128, 128)], add=True)

            def relu_row(r, c2):
                evr[r, pl.ds(0, 16)] = jnp.maximum(evr[r, pl.ds(0, 16)], 0.0)
                evr[r, pl.ds(16, 16)] = jnp.maximum(evr[r, pl.ds(16, 16)], 0.0)
                return c2

            lax.fori_loop(0, _B, relu_row, 0, unroll=4)
            for j in range(_B // 128):
                pltpu.sync_copy(evr.at[pl.ds(j * 128, 128)],
                                acc.at[dv.at[j]], add=True)

        issue(0, sva, dva, eva, sema)

        def pair(g, carry):
            i0 = g * 2
            issue(i0 + 1, svb, dvb, evb, semb)
            process(i0, sva, dva, eva, sema)

            @pl.when(g + 1 < _NB // 2)
            def _():
                issue(i0 + 2, sva, dva, eva, sema)

            process(i0 + 1, svb, dvb, evb, semb)
            return carry

        lax.fori_loop(0, _NB // 2, pair, 0)
        plsc.subcore_barrier()

        @pl.when(c == 0)
        def _():
            @pl.when(s < 15)
            def _():
                pltpu.sync_copy(acc.at[pl.ds(s * _RPT, _RPT)],
                                outa_hbm.at[pl.ds(s * _RPT, _RPT)])

            @pl.when(s == 15)
            def _():
                pltpu.sync_copy(acc.at[pl.ds(15 * _RPT, _N - 15 * _RPT)],
                                outa_hbm.at[pl.ds(15 * _RPT, _N - 15 * _RPT)])

        @pl.when(c == 1)
        def _():
            @pl.when(s < 15)
            def _():
                pltpu.sync_copy(acc.at[pl.ds(s * _RPT, _RPT)],
                                outb_hbm.at[pl.ds(s * _RPT, _RPT)])

            @pl.when(s == 15)
            def _():
                pltpu.sync_copy(acc.at[pl.ds(15 * _RPT, _N - 15 * _RPT)],
                                outb_hbm.at[pl.ds(15 * _RPT, _N - 15 * _RPT)])

    f = pl.kernel(
        body,
        out_type=[jax.ShapeDtypeStruct((_N, _HH), jnp.float32),
                  jax.ShapeDtypeStruct((_N, _HH), jnp.float32)],
        mesh=mesh,
        compiler_params=pltpu.CompilerParams(use_tc_tiling_on_sc=False),
        scratch_types=[
            pltpu.VMEM((_B,), jnp.int32),
            pltpu.VMEM((_B // 128, 128), jnp.int32),
            pltpu.VMEM((_B, _HH), jnp.float32),
            pltpu.VMEM((_B,), jnp.int32),
            pltpu.VMEM((_B // 128, 128), jnp.int32),
            pltpu.VMEM((_B, _HH), jnp.float32),
            pltpu.SemaphoreType.DMA,
            pltpu.SemaphoreType.DMA,
            pltpu.VMEM_SHARED((_NP, _HH), jnp.float32),
        ],
    )
    return f(ha, hb, e2.reshape(2 * _EPAD, _HH), src_p, dst2, zrows)


def kernel(x, pre_x, edge_index, edge_attr, batch,
           W_node, b_node, W_nn, b_nn,
           W_edge0, b_edge0, W_edge1, b_edge1, W_edge2, b_edge2,
           W1, b1, W2, b2):
    npad = _EPAD - _E
    src_p = jnp.concatenate([edge_index[0], jnp.zeros((npad,), jnp.int32)])
    dst_p = jnp.concatenate([edge_index[1],
                             jnp.arange(npad, dtype=jnp.int32)])
    dst2 = dst_p.reshape(_EPAD // 128, 128)
    attr4 = edge_attr.reshape(_E // 4, 64)
    zrows = jnp.zeros((_RPT, _HH), jnp.float32)

    ha, hb = _node_embed(x, pre_x, W_node, b_node)
    edge_lins = ((W_edge0, b_edge0), (W_edge1, b_edge1), (W_edge2, b_edge2))
    e2s = [_edge_proj(attr4, We, be) for We, be in edge_lins]
    for e2 in e2s:
        aa, ab = _sc_aggr(ha, hb, e2, src_p, dst2, zrows)
        ha, hb = _layer_mlp(ha, hb, aa, ab, W_nn, b_nn)
    return _readout(ha, hb, batch, W1, b1, W2, b2)


# speedup vs baseline: 3.2927x; 1.1917x over previous
import functools

import jax
import jax.numpy as jnp
from jax import lax
from jax.experimental import pallas as pl
from jax.experimental.pallas import tpu as pltpu
from jax.experimental.pallas import tpu_sc as plsc

_N = 50000
_E = 800000
_EPAD = 819200
_EPT = _EPAD // 16
_B = 256
_NB = _EPT // _B
_RPT = 3128
_NP = 16 * _RPT
_H = 64
_HH = 32
_BN = 400
_GN = _N // _BN
_BE4 = 1600
_GE4 = _EPAD // 4 // _BE4
_RE4 = _E // 4 // _BE4
_NG = 256
_BNR = 2000
_PREC = lax.Precision.HIGHEST


def _embed_body(x_ref, p_ref, wx_ref, wp_ref, b_ref, oa_ref, ob_ref):
    h = (jnp.dot(x_ref[...], wx_ref[...], precision=_PREC,
                 preferred_element_type=jnp.float32)
         + jnp.dot(p_ref[...], wp_ref[...], precision=_PREC,
                   preferred_element_type=jnp.float32)
         + b_ref[...])
    oa_ref[...] = h[:, :_HH]
    ob_ref[...] = h[:, _HH:]


def _node_embed(x, pre_x, W_node, b_node):
    wx = W_node[:128]
    wp = W_node[128:]
    return pl.pallas_call(
        _embed_body,
        grid=(_GN,),
        in_specs=[
            pl.BlockSpec((_BN, 128), lambda i: (i, 0)),
            pl.BlockSpec((_BN, 256), lambda i: (i, 0)),
            pl.BlockSpec((128, _H), lambda i: (0, 0)),
            pl.BlockSpec((256, _H), lambda i: (0, 0)),
            pl.BlockSpec((_H,), lambda i: (0,)),
        ],
        out_specs=[
            pl.BlockSpec((_BN, _HH), lambda i: (i, 0)),
            pl.BlockSpec((_BN, _HH), lambda i: (i, 0)),
        ],
        out_shape=[
            jax.ShapeDtypeStruct((_N, _HH), jnp.float32),
            jax.ShapeDtypeStruct((_N, _HH), jnp.float32),
        ],
    )(x, pre_x, wx, wp, b_node)


def _eproj_body(a_ref, w_ref, b_ref, o_ref):
    i = pl.program_id(1)
    e = jnp.dot(a_ref[...], w_ref[0],
                preferred_element_type=jnp.float32) + b_ref[0, 0]
    o_ref[...] = jnp.where(i < _RE4, e, jnp.float32(-1e9))


def _edge_proj(attr4, We, be):
    eye4 = jnp.eye(4, dtype=jnp.float32)
    w4 = jnp.stack([jnp.kron(eye4, We[:, :_HH]), jnp.kron(eye4, We[:, _HH:])])
    b4 = jnp.stack([jnp.tile(be[:_HH], 4), jnp.tile(be[_HH:], 4)])
    b4 = b4.reshape(2, 1, 128)
    return pl.pallas_call(
        _eproj_body,
        grid=(2, _GE4),
        in_specs=[
            pl.BlockSpec((_BE4, 64), lambda c, i: (jnp.minimum(i, _RE4 - 1), 0)),
            pl.BlockSpec((1, 64, 128), lambda c, i: (c, 0, 0)),
            pl.BlockSpec((1, 1, 128), lambda c, i: (c, 0, 0)),
        ],
        out_specs=pl.BlockSpec((_BE4, 128), lambda c, i: (c * _GE4 + i, 0)),
        out_shape=jax.ShapeDtypeStruct((2 * _EPAD // 4, 128), jnp.float32),
    )(attr4, w4, b4)


def _layer_body(ha_ref, hb_ref, aa_ref, ab_ref, wa_ref, wb_ref, b_ref,
                oa_ref, ob_ref):
    za = ha_ref[...] + aa_ref[...]
    zb = hb_ref[...] + ab_ref[...]
    h = (jnp.dot(za, wa_ref[...], precision=_PREC,
                 preferred_element_type=jnp.float32)
         + jnp.dot(zb, wb_ref[...], precision=_PREC,
                   preferred_element_type=jnp.float32)
         + b_ref[...])
    h = jnp.maximum(h, 0.0)
    oa_ref[...] = h[:, :_HH]
    ob_ref[...] = h[:, _HH:]


def _layer_mlp(ha, hb, aa, ab, W_nn, b_nn):
    wa = W_nn[:_HH]
    wb = W_nn[_HH:]
    return pl.pallas_call(
        _layer_body,
        grid=(_GN,),
        in_specs=[
            pl.BlockSpec((_BN, _HH), lambda i: (i, 0)),
            pl.BlockSpec((_BN, _HH), lambda i: (i, 0)),
            pl.BlockSpec((_BN, _HH), lambda i: (i, 0)),
            pl.BlockSpec((_BN, _HH), lambda i: (i, 0)),
            pl.BlockSpec((_HH, _H), lambda i: (0, 0)),
            pl.BlockSpec((_HH, _H), lambda i: (0, 0)),
            pl.BlockSpec((_H,), lambda i: (0,)),
        ],
        out_specs=[
            pl.BlockSpec((_BN, _HH), lambda i: (i, 0)),
            pl.BlockSpec((_BN, _HH), lambda i: (i, 0)),
        ],
        out_shape=[
            jax.ShapeDtypeStruct((_N, _HH), jnp.float32),
            jax.ShapeDtypeStruct((_N, _HH), jnp.float32),
        ],
    )(ha, hb, aa, ab, wa, wb, b_nn)


def _readout_body(ha_ref, hb_ref, bt_ref, w1a_ref, w1b_ref, b1_ref,
                  w2_ref, b2_ref, o_ref):
    i = pl.program_id(0)

    @pl.when(i == 0)
    def _():
        o_ref[...] = jnp.zeros_like(o_ref)

    t = (jnp.dot(ha_ref[...], w1a_ref[...],
                 preferred_element_type=jnp.float32)
         + jnp.dot(hb_ref[...], w1b_ref[...],
                   preferred_element_type=jnp.float32)
         + b1_ref[...])
    t = jnp.maximum(t, 0.0)
    y = jnp.dot(t, w2_ref[...],
                preferred_element_type=jnp.float32) + b2_ref[...]
    bt = bt_ref[0, 0, :]
    onehot = (bt[:, None] ==
              lax.broadcasted_iota(jnp.int32, (1, _NG), 1)).astype(jnp.float32)
    o_ref[...] += lax.dot_general(onehot, y, (((0,), (0,)), ((), ())),
                                  precision=_PREC,
                                  preferred_element_type=jnp.float32)


def _readout(ha, hb, batch, W1, b1, W2, b2):
    w1a = W1[:_HH]
    w1b = W1[_HH:]
    bt3 = batch.reshape(_N // _BNR, 1, _BNR)
    return pl.pallas_call(
        _readout_body,
        grid=(_N // _BNR,),
        in_specs=[
            pl.BlockSpec((_BNR, _HH), lambda i: (i, 0)),
            pl.BlockSpec((_BNR, _HH), lambda i: (i, 0)),
            pl.BlockSpec((1, 1, _BNR), lambda i: (i, 0, 0)),
            pl.BlockSpec((_HH, 1024), lambda i: (0, 0)),
            pl.BlockSpec((_HH, 1024), lambda i: (0, 0)),
            pl.BlockSpec((1024,), lambda i: (0,)),
            pl.BlockSpec((1024, 128), lambda i: (0, 0)),
            pl.BlockSpec((128,), lambda i: (0,)),
        ],
        out_specs=pl.BlockSpec((_NG, 128), lambda i: (0, 0)),
        out_shape=jax.ShapeDtypeStruct((_NG, 128), jnp.float32),
    )(ha, hb, bt3, w1a, w1b, b1, W2, b2)


def _sc_aggr(ha, hb, e2, src_p, dst2, zrows):
    mesh = plsc.VectorSubcoreMesh(core_axis_name="c", subcore_axis_name="s")

    def body(ha_hbm, hb_hbm, e_hbm, src_hbm, dst_hbm, z_hbm,
             outa_hbm, outb_hbm,
             sv0, dv0, ev0, sv1, dv1, ev1, sv2, dv2, ev2,
             lsem0, lsem1, lsem2, ssem0, ssem1, ssem2, gsem, acc):
        c = lax.axis_index("c")
        s = lax.axis_index("s")
        pltpu.sync_copy(z_hbm, acc.at[pl.ds(s * _RPT, _RPT)])
        plsc.subcore_barrier()
        ebase0 = s * _EPT
        slots = ((sv0, dv0, ev0, lsem0, ssem0),
                 (sv1, dv1, ev1, lsem1, ssem1),
                 (sv2, dv2, ev2, lsem2, ssem2))

        def slices(i):
            base = ebase0 + i * _B
            rbase = s * (_EPT // 128) + i * (_B // 128)
            return (src_hbm.at[pl.ds(base, _B)],
                    dst_hbm.at[pl.ds(rbase, _B // 128)],
                    e_hbm.at[pl.ds(c * _EPAD + base, _B)])

        def issue(i, sv, dv, evr, lsem):
            ss, ds_, es = slices(i)
            pltpu.async_copy(ss, sv, lsem)
            pltpu.async_copy(ds_, dv, lsem)
            pltpu.async_copy(es, evr, lsem)

        def drain_scatter(dv, evr, ssem):
            for j in range(_B // 128):
                pltpu.make_async_copy(evr.at[pl.ds(j * 128, 128)],
                                      acc.at[dv.at[j]], ssem).wait()

        def process(i, sv, dv, evr, lsem, ssem):
            ss, ds_, es = slices(i)
            pltpu.make_async_copy(ss, sv, lsem).wait()
            pltpu.make_async_copy(ds_, dv, lsem).wait()
            pltpu.make_async_copy(es, evr, lsem).wait()

            @pl.when(c == 0)
            def _():
                for j in range(_B // 128):
                    pltpu.async_copy(ha_hbm.at[sv.at[pl.ds(j * 128, 128)]],
                                     evr.at[pl.ds(j * 128, 128)], gsem,
                                     add=True)

            @pl.when(c == 1)
            def _():
                for j in range(_B // 128):
                    pltpu.async_copy(hb_hbm.at[sv.at[pl.ds(j * 128, 128)]],
                                     evr.at[pl.ds(j * 128, 128)], gsem,
                                     add=True)

            for j in range(_B // 128):
                pltpu.make_async_copy(ha_hbm.at[sv.at[pl.ds(j * 128, 128)]],
                                      evr.at[pl.ds(j * 128, 128)],
                                      gsem).wait()

            def relu_row(r, c2):
                evr[r, pl.ds(0, 16)] = jnp.maximum(evr[r, pl.ds(0, 16)], 0.0)
                evr[r, pl.ds(16, 16)] = jnp.maximum(evr[r, pl.ds(16, 16)], 0.0)
                return c2

            lax.fori_loop(0, _B, relu_row, 0, unroll=4)
            for j in range(_B // 128):
                pltpu.async_copy(evr.at[pl.ds(j * 128, 128)],
                                 acc.at[dv.at[j]], ssem, add=True)

        issue(0, *slots[0][:3], slots[0][3])
        issue(1, *slots[1][:3], slots[1][3])

        def triple(g, carry):
            i0 = g * 3
            for q in range(3):
                sv, dv, evr, lsem, ssem = slots[q]
                process(i0 + q, sv, dv, evr, lsem, ssem)
                nsv, ndv, nev, nls, nss = slots[(q + 2) % 3]
                if q == 0:
                    @pl.when(g > 0)
                    def _():
                        drain_scatter(ndv, nev, nss)
                else:
                    drain_scatter(ndv, nev, nss)
                issue(i0 + q + 2, nsv, ndv, nev, nls)
            return carry

        lax.fori_loop(0, _NB // 3, triple, 0)
        _i = (_NB // 3) * 3
        process(_i, *slots[0])
        process(_i + 1, *slots[1])
        drain_scatter(dv2, ev2, ssem2)
        drain_scatter(dv0, ev0, ssem0)
        drain_scatter(dv1, ev1, ssem1)
        plsc.subcore_barrier()

        @pl.when(c == 0)
        def _():
            @pl.when(s < 15)
            def _():
                pltpu.sync_copy(acc.at[pl.ds(s * _RPT, _RPT)],
                                outa_hbm.at[pl.ds(s * _RPT, _RPT)])

            @pl.when(s == 15)
            def _():
                pltpu.sync_copy(acc.at[pl.ds(15 * _RPT, _N - 15 * _RPT)],
                                outa_hbm.at[pl.ds(15 * _RPT, _N - 15 * _RPT)])

        @pl.when(c == 1)
        def _():
            @pl.when(s < 15)
            def _():
                pltpu.sync_copy(acc.at[pl.ds(s * _RPT, _RPT)],
                                outb_hbm.at[pl.ds(s * _RPT, _RPT)])

            @pl.when(s == 15)
            def _():
                pltpu.sync_copy(acc.at[pl.ds(15 * _RPT, _N - 15 * _RPT)],
                                outb_hbm.at[pl.ds(15 * _RPT, _N - 15 * _RPT)])

    f = pl.kernel(
        body,
        out_type=[jax.ShapeDtypeStruct((_N, _HH), jnp.float32),
                  jax.ShapeDtypeStruct((_N, _HH), jnp.float32)],
        mesh=mesh,
        compiler_params=pltpu.CompilerParams(use_tc_tiling_on_sc=False),
        scratch_types=(
            [pltpu.VMEM((_B,), jnp.int32),
             pltpu.VMEM((_B // 128, 128), jnp.int32),
             pltpu.VMEM((_B, _HH), jnp.float32)] * 3
            + [pltpu.SemaphoreType.DMA] * 7
            + [pltpu.VMEM_SHARED((_NP, _HH), jnp.float32)]
        ),
    )
    return f(ha, hb, e2.reshape(2 * _EPAD, _HH), src_p, dst2, zrows)


def kernel(x, pre_x, edge_index, edge_attr, batch,
           W_node, b_node, W_nn, b_nn,
           W_edge0, b_edge0, W_edge1, b_edge1, W_edge2, b_edge2,
           W1, b1, W2, b2):
    npad = _EPAD - _E
    src_p = jnp.concatenate([edge_index[0], jnp.zeros((npad,), jnp.int32)])
    dst_p = jnp.concatenate([edge_index[1],
                             jnp.arange(npad, dtype=jnp.int32)])
    dst2 = dst_p.reshape(_EPAD // 128, 128)
    attr4 = edge_attr.reshape(_E // 4, 64)
    zrows = jnp.zeros((_RPT, _HH), jnp.float32)

    ha, hb = _node_embed(x, pre_x, W_node, b_node)
    edge_lins = ((W_edge0, b_edge0), (W_edge1, b_edge1), (W_edge2, b_edge2))
    e2s = [_edge_proj(attr4, We, be) for We, be in edge_lins]
    for e2 in e2s:
        aa, ab = _sc_aggr(ha, hb, e2, src_p, dst2, zrows)
        ha, hb = _layer_mlp(ha, hb, aa, ab, W_nn, b_nn)
    return _readout(ha, hb, batch, W1, b1, W2, b2)


# speedup vs baseline: 3.6225x; 1.1001x over previous
import functools

import jax
import jax.numpy as jnp
from jax import lax
from jax.experimental import pallas as pl
from jax.experimental.pallas import tpu as pltpu
from jax.experimental.pallas import tpu_sc as plsc

_N = 50000
_E = 800000
_EPAD = 819200
_EPT = _EPAD // 16
_B = 256
_NB = _EPT // _B
_RPT = 3128
_NP = 16 * _RPT
_H = 64
_HH = 32
_BN = 2000
_GN = _N // _BN
_BE4 = 1600
_GE4 = _EPAD // 4 // _BE4
_RE4 = _E // 4 // _BE4
_NG = 256
_BNR = 2000
_PREC = lax.Precision.HIGHEST


def _embed_body(x_ref, p_ref, wx_ref, wp_ref, b_ref, oa_ref, ob_ref):
    h = (jnp.dot(x_ref[...], wx_ref[...], precision=_PREC,
                 preferred_element_type=jnp.float32)
         + jnp.dot(p_ref[...], wp_ref[...], precision=_PREC,
                   preferred_element_type=jnp.float32)
         + b_ref[...])
    oa_ref[...] = h[:, :_HH]
    ob_ref[...] = h[:, _HH:]


def _node_embed(x, pre_x, W_node, b_node):
    wx = W_node[:128]
    wp = W_node[128:]
    return pl.pallas_call(
        _embed_body,
        grid=(_GN,),
        in_specs=[
            pl.BlockSpec((_BN, 128), lambda i: (i, 0)),
            pl.BlockSpec((_BN, 256), lambda i: (i, 0)),
            pl.BlockSpec((128, _H), lambda i: (0, 0)),
            pl.BlockSpec((256, _H), lambda i: (0, 0)),
            pl.BlockSpec((_H,), lambda i: (0,)),
        ],
        out_specs=[
            pl.BlockSpec((_BN, _HH), lambda i: (i, 0)),
            pl.BlockSpec((_BN, _HH), lambda i: (i, 0)),
        ],
        out_shape=[
            jax.ShapeDtypeStruct((_N, _HH), jnp.float32),
            jax.ShapeDtypeStruct((_N, _HH), jnp.float32),
        ],
    )(x, pre_x, wx, wp, b_node)


def _eproj_body(a_ref, w_ref, b_ref, o_ref):
    i = pl.program_id(1)
    e = jnp.dot(a_ref[...], w_ref[0],
                preferred_element_type=jnp.float32) + b_ref[0, 0]
    o_ref[...] = jnp.where(i < _RE4, e, jnp.float32(-1e9))


def _edge_proj(attr4, We, be):
    w4 = jnp.stack([
        jnp.concatenate([jnp.pad(We[:, h * _HH:(h + 1) * _HH],
                                 ((0, 0), (q * _HH, (3 - q) * _HH)))
                         for q in range(4)], axis=0)
        for h in range(2)])
    b4 = jnp.stack([jnp.tile(be[:_HH], 4), jnp.tile(be[_HH:], 4)])
    b4 = b4.reshape(2, 1, 128)
    return pl.pallas_call(
        _eproj_body,
        grid=(2, _GE4),
        in_specs=[
            pl.BlockSpec((_BE4, 64), lambda c, i: (jnp.minimum(i, _RE4 - 1), 0)),
            pl.BlockSpec((1, 64, 128), lambda c, i: (c, 0, 0)),
            pl.BlockSpec((1, 1, 128), lambda c, i: (c, 0, 0)),
        ],
        out_specs=pl.BlockSpec((_BE4, 128), lambda c, i: (c * _GE4 + i, 0)),
        out_shape=jax.ShapeDtypeStruct((2 * _EPAD // 4, 128), jnp.float32),
    )(attr4, w4, b4)


def _layer_body(ha_ref, hb_ref, aa_ref, ab_ref, wa_ref, wb_ref, b_ref,
                oa_ref, ob_ref):
    za = ha_ref[...] + aa_ref[...]
    zb = hb_ref[...] + ab_ref[...]
    h = (jnp.dot(za, wa_ref[...], precision=_PREC,
                 preferred_element_type=jnp.float32)
         + jnp.dot(zb, wb_ref[...], precision=_PREC,
                   preferred_element_type=jnp.float32)
         + b_ref[...])
    h = jnp.maximum(h, 0.0)
    oa_ref[...] = h[:, :_HH]
    ob_ref[...] = h[:, _HH:]


def _layer_mlp(ha, hb, aa, ab, W_nn, b_nn):
    wa = W_nn[:_HH]
    wb = W_nn[_HH:]
    return pl.pallas_call(
        _layer_body,
        grid=(_GN,),
        in_specs=[
            pl.BlockSpec((_BN, _HH), lambda i: (i, 0)),
            pl.BlockSpec((_BN, _HH), lambda i: (i, 0)),
            pl.BlockSpec((_BN, _HH), lambda i: (i, 0)),
            pl.BlockSpec((_BN, _HH), lambda i: (i, 0)),
            pl.BlockSpec((_HH, _H), lambda i: (0, 0)),
            pl.BlockSpec((_HH, _H), lambda i: (0, 0)),
            pl.BlockSpec((_H,), lambda i: (0,)),
        ],
        out_specs=[
            pl.BlockSpec((_BN, _HH), lambda i: (i, 0)),
            pl.BlockSpec((_BN, _HH), lambda i: (i, 0)),
        ],
        out_shape=[
            jax.ShapeDtypeStruct((_N, _HH), jnp.float32),
            jax.ShapeDtypeStruct((_N, _HH), jnp.float32),
        ],
    )(ha, hb, aa, ab, wa, wb, b_nn)


def _readout_body(ha_ref, hb_ref, bt_ref, w1a_ref, w1b_ref, b1_ref,
                  w2_ref, b2_ref, o_ref):
    i = pl.program_id(0)

    @pl.when(i == 0)
    def _():
        o_ref[...] = jnp.zeros_like(o_ref)

    t = (jnp.dot(ha_ref[...], w1a_ref[...],
                 preferred_element_type=jnp.float32)
         + jnp.dot(hb_ref[...], w1b_ref[...],
                   preferred_element_type=jnp.float32)
         + b1_ref[...])
    t = jnp.maximum(t, 0.0)
    y = jnp.dot(t, w2_ref[...],
                preferred_element_type=jnp.float32) + b2_ref[...]
    bt = bt_ref[0, 0, :]
    onehot = (bt[:, None] ==
              lax.broadcasted_iota(jnp.int32, (1, _NG), 1)).astype(jnp.float32)
    o_ref[...] += lax.dot_general(onehot, y, (((0,), (0,)), ((), ())),
                                  precision=_PREC,
                                  preferred_element_type=jnp.float32)


def _readout(ha, hb, batch, W1, b1, W2, b2):
    w1a = W1[:_HH]
    w1b = W1[_HH:]
    bt3 = batch.reshape(_N // _BNR, 1, _BNR)
    return pl.pallas_call(
        _readout_body,
        grid=(_N // _BNR,),
        in_specs=[
            pl.BlockSpec((_BNR, _HH), lambda i: (i, 0)),
            pl.BlockSpec((_BNR, _HH), lambda i: (i, 0)),
            pl.BlockSpec((1, 1, _BNR), lambda i: (i, 0, 0)),
            pl.BlockSpec((_HH, 1024), lambda i: (0, 0)),
            pl.BlockSpec((_HH, 1024), lambda i: (0, 0)),
            pl.BlockSpec((1024,), lambda i: (0,)),
            pl.BlockSpec((1024, 128), lambda i: (0, 0)),
            pl.BlockSpec((128,), lambda i: (0,)),
        ],
        out_specs=pl.BlockSpec((_NG, 128), lambda i: (0, 0)),
        out_shape=jax.ShapeDtypeStruct((_NG, 128), jnp.float32),
    )(ha, hb, bt3, w1a, w1b, b1, W2, b2)


def _sc_aggr(ha, hb, e2, src_p, dst2, zrows):
    mesh = plsc.VectorSubcoreMesh(core_axis_name="c", subcore_axis_name="s")

    def body(ha_hbm, hb_hbm, e_hbm, src_hbm, dst_hbm, z_hbm,
             outa_hbm, outb_hbm,
             sv0, dv0, ev0, sv1, dv1, ev1, sv2, dv2, ev2,
             lsem0, lsem1, lsem2, ssem0, ssem1, ssem2, gsem, acc):
        c = lax.axis_index("c")
        s = lax.axis_index("s")
        pltpu.sync_copy(z_hbm, acc.at[pl.ds(s * _RPT, _RPT)])
        plsc.subcore_barrier()
        ebase0 = s * _EPT
        slots = ((sv0, dv0, ev0, lsem0, ssem0),
                 (sv1, dv1, ev1, lsem1, ssem1),
                 (sv2, dv2, ev2, lsem2, ssem2))

        def slices(i):
            base = ebase0 + i * _B
            rbase = s * (_EPT // 128) + i * (_B // 128)
            return (src_hbm.at[pl.ds(base, _B)],
                    dst_hbm.at[pl.ds(rbase, _B // 128)],
                    e_hbm.at[pl.ds(c * _EPAD + base, _B)])

        def issue(i, sv, dv, evr, lsem):
            ss, ds_, es = slices(i)
            pltpu.async_copy(ss, sv, lsem)
            pltpu.async_copy(ds_, dv, lsem)
            pltpu.async_copy(es, evr, lsem)

        def drain_scatter(dv, evr, ssem):
            for j in range(_B // 128):
                pltpu.make_async_copy(evr.at[pl.ds(j * 128, 128)],
                                      acc.at[dv.at[j]], ssem).wait()

        def process(i, sv, dv, evr, lsem, ssem):
            ss, ds_, es = slices(i)
            pltpu.make_async_copy(ss, sv, lsem).wait()
            pltpu.make_async_copy(ds_, dv, lsem).wait()
            pltpu.make_async_copy(es, evr, lsem).wait()

            @pl.when(c == 0)
            def _():
                for j in range(_B // 128):
                    pltpu.async_copy(ha_hbm.at[sv.at[pl.ds(j * 128, 128)]],
                                     evr.at[pl.ds(j * 128, 128)], gsem,
                                     add=True)

            @pl.when(c == 1)
            def _():
                for j in range(_B // 128):
                    pltpu.async_copy(hb_hbm.at[sv.at[pl.ds(j * 128, 128)]],
                                     evr.at[pl.ds(j * 128, 128)], gsem,
                                     add=True)

            for j in range(_B // 128):
                pltpu.make_async_copy(ha_hbm.at[sv.at[pl.ds(j * 128, 128)]],
                                      evr.at[pl.ds(j * 128, 128)],
                                      gsem).wait()

            def relu_row(r, c2):
                evr[r, pl.ds(0, 16)] = jnp.maximum(evr[r, pl.ds(0, 16)], 0.0)
                evr[r, pl.ds(16, 16)] = jnp.maximum(evr[r, pl.ds(16, 16)], 0.0)
                return c2

            lax.fori_loop(0, _B, relu_row, 0, unroll=4)
            for j in range(_B // 128):
                pltpu.async_copy(evr.at[pl.ds(j * 128, 128)],
                                 acc.at[dv.at[j]], ssem, add=True)

        issue(0, *slots[0][:3], slots[0][3])
        issue(1, *slots[1][:3], slots[1][3])

        def triple(g, carry):
            i0 = g * 3
            for q in range(3):
                sv, dv, evr, lsem, ssem = slots[q]
                process(i0 + q, sv, dv, evr, lsem, ssem)
                nsv, ndv, nev, nls, nss = slots[(q + 2) % 3]
                if q == 0:
                    @pl.when(g > 0)
                    def _():
                        drain_scatter(ndv, nev, nss)
                else:
                    drain_scatter(ndv, nev, nss)
                issue(i0 + q + 2, nsv, ndv, nev, nls)
            return carry

        lax.fori_loop(0, _NB // 3, triple, 0)
        _i = (_NB // 3) * 3
        process(_i, *slots[0])
        process(_i + 1, *slots[1])
        drain_scatter(dv2, ev2, ssem2)
        drain_scatter(dv0, ev0, ssem0)
        drain_scatter(dv1, ev1, ssem1)
        plsc.subcore_barrier()

        @pl.when(c == 0)
        def _():
            @pl.when(s < 15)
            def _():
                pltpu.sync_copy(acc.at[pl.ds(s * _RPT, _RPT)],
                                outa_hbm.at[pl.ds(s * _RPT, _RPT)])

            @pl.when(s == 15)
            def _():
                pltpu.sync_copy(acc.at[pl.ds(15 * _RPT, _N - 15 * _RPT)],
                                outa_hbm.at[pl.ds(15 * _RPT, _N - 15 * _RPT)])

        @pl.when(c == 1)
        def _():
            @pl.when(s < 15)
            def _():
                pltpu.sync_copy(acc.at[pl.ds(s * _RPT, _RPT)],
                                outb_hbm.at[pl.ds(s * _RPT, _RPT)])

            @pl.when(s == 15)
            def _():
                pltpu.sync_copy(acc.at[pl.ds(15 * _RPT, _N - 15 * _RPT)],
                                outb_hbm.at[pl.ds(15 * _RPT, _N - 15 * _RPT)])

    f = pl.kernel(
        body,
        out_type=[jax.ShapeDtypeStruct((_N, _HH), jnp.float32),
                  jax.ShapeDtypeStruct((_N, _HH), jnp.float32)],
        mesh=mesh,
        compiler_params=pltpu.CompilerParams(use_tc_tiling_on_sc=False),
        scratch_types=(
            [pltpu.VMEM((_B,), jnp.int32),
             pltpu.VMEM((_B // 128, 128), jnp.int32),
             pltpu.VMEM((_B, _HH), jnp.float32)] * 3
            + [pltpu.SemaphoreType.DMA] * 7
            + [pltpu.VMEM_SHARED((_NP, _HH), jnp.float32)]
        ),
    )
    return f(ha, hb, e2.reshape(2 * _EPAD, _HH), src_p, dst2, zrows)


def kernel(x, pre_x, edge_index, edge_attr, batch,
           W_node, b_node, W_nn, b_nn,
           W_edge0, b_edge0, W_edge1, b_edge1, W_edge2, b_edge2,
           W1, b1, W2, b2):
    npad = _EPAD - _E
    src_p = jnp.concatenate([edge_index[0], jnp.zeros((npad,), jnp.int32)])
    dst_p = jnp.concatenate([edge_index[1],
                             jnp.arange(npad, dtype=jnp.int32)])
    dst2 = dst_p.reshape(_EPAD // 128, 128)
    zrows = jnp.zeros((_RPT, _HH), jnp.float32)

    ha, hb = _node_embed(x, pre_x, W_node, b_node)
    edge_lins = ((W_edge0, b_edge0), (W_edge1, b_edge1), (W_edge2, b_edge2))
    attr4 = edge_attr.reshape(_E // 4, 64)
    e2s = [_edge_proj(attr4, We, be) for We, be in edge_lins]
    for e2 in e2s:
        aa, ab = _sc_aggr(ha, hb, e2, src_p, dst2, zrows)
        ha, hb = _layer_mlp(ha, hb, aa, ab, W_nn, b_nn)
    return _readout(ha, hb, batch, W1, b1, W2, b2)


# speedup vs baseline: 3.8326x; 1.0580x over previous
import functools

import jax
import jax.numpy as jnp
from jax import lax
from jax.experimental import pallas as pl
from jax.experimental.pallas import tpu as pltpu
from jax.experimental.pallas import tpu_sc as plsc

_N = 50000
_E = 800000
_EPAD = 819200
_EPT = _EPAD // 16
_B = 256
_NB = _EPT // _B
_RPT = 3128
_NP = 16 * _RPT
_H = 64
_HH = 32
_BN = 2000
_GN = _N // _BN
_BE4 = 1600
_GE4 = _EPAD // 4 // _BE4
_RE4 = _E // 4 // _BE4
_NG = 256
_BNR = 2000
_PREC = lax.Precision.HIGHEST


def _embed_body(x_ref, p_ref, wx_ref, wp_ref, b_ref, oa_ref, ob_ref):
    h = (jnp.dot(x_ref[...], wx_ref[...], precision=_PREC,
                 preferred_element_type=jnp.float32)
         + jnp.dot(p_ref[...], wp_ref[...], precision=_PREC,
                   preferred_element_type=jnp.float32)
         + b_ref[...])
    oa_ref[...] = h[:, :_HH]
    ob_ref[...] = h[:, _HH:]


def _node_embed(x, pre_x, W_node, b_node):
    wx = W_node[:128]
    wp = W_node[128:]
    return pl.pallas_call(
        _embed_body,
        grid=(_GN,),
        in_specs=[
            pl.BlockSpec((_BN, 128), lambda i: (i, 0)),
            pl.BlockSpec((_BN, 256), lambda i: (i, 0)),
            pl.BlockSpec((128, _H), lambda i: (0, 0)),
            pl.BlockSpec((256, _H), lambda i: (0, 0)),
            pl.BlockSpec((_H,), lambda i: (0,)),
        ],
        out_specs=[
            pl.BlockSpec((_BN, _HH), lambda i: (i, 0)),
            pl.BlockSpec((_BN, _HH), lambda i: (i, 0)),
        ],
        out_shape=[
            jax.ShapeDtypeStruct((_N, _HH), jnp.float32),
            jax.ShapeDtypeStruct((_N, _HH), jnp.float32),
        ],
    )(x, pre_x, wx, wp, b_node)


def _eproj_body(a_ref, w_ref, b_ref, o_ref):
    i = pl.program_id(1)
    e = jnp.dot(a_ref[...], w_ref[0],
                preferred_element_type=jnp.float32) + b_ref[0, 0]
    o_ref[...] = jnp.where(i < _RE4, e, jnp.float32(-1e9))


def _edge_proj(attr4, We, be):
    w4 = jnp.stack([
        jnp.concatenate([jnp.pad(We[:, h * _HH:(h + 1) * _HH],
                                 ((0, 0), (q * _HH, (3 - q) * _HH)))
                         for q in range(4)], axis=0)
        for h in range(2)])
    b4 = jnp.stack([jnp.tile(be[:_HH], 4), jnp.tile(be[_HH:], 4)])
    b4 = b4.reshape(2, 1, 128)
    return pl.pallas_call(
        _eproj_body,
        grid=(2, _GE4),
        in_specs=[
            pl.BlockSpec((_BE4, 64), lambda c, i: (jnp.minimum(i, _RE4 - 1), 0)),
            pl.BlockSpec((1, 64, 128), lambda c, i: (c, 0, 0)),
            pl.BlockSpec((1, 1, 128), lambda c, i: (c, 0, 0)),
        ],
        out_specs=pl.BlockSpec((_BE4, 128), lambda c, i: (c * _GE4 + i, 0)),
        out_shape=jax.ShapeDtypeStruct((2 * _EPAD // 4, 128), jnp.float32),
    )(attr4, w4, b4)


def _layer_body(ha_ref, hb_ref, aa_ref, ab_ref, wa_ref, wb_ref, b_ref,
                oa_ref, ob_ref):
    za = ha_ref[...] + aa_ref[...]
    zb = hb_ref[...] + ab_ref[...]
    h = (jnp.dot(za, wa_ref[...], precision=_PREC,
                 preferred_element_type=jnp.float32)
         + jnp.dot(zb, wb_ref[...], precision=_PREC,
                   preferred_element_type=jnp.float32)
         + b_ref[...])
    h = jnp.maximum(h, 0.0)
    oa_ref[...] = h[:, :_HH]
    ob_ref[...] = h[:, _HH:]


def _layer_mlp(ha, hb, aa, ab, W_nn, b_nn):
    wa = W_nn[:_HH]
    wb = W_nn[_HH:]
    return pl.pallas_call(
        _layer_body,
        grid=(_GN,),
        in_specs=[
            pl.BlockSpec((_BN, _HH), lambda i: (i, 0)),
            pl.BlockSpec((_BN, _HH), lambda i: (i, 0)),
            pl.BlockSpec((_BN, _HH), lambda i: (i, 0)),
            pl.BlockSpec((_BN, _HH), lambda i: (i, 0)),
            pl.BlockSpec((_HH, _H), lambda i: (0, 0)),
            pl.BlockSpec((_HH, _H), lambda i: (0, 0)),
            pl.BlockSpec((_H,), lambda i: (0,)),
        ],
        out_specs=[
            pl.BlockSpec((_BN, _HH), lambda i: (i, 0)),
            pl.BlockSpec((_BN, _HH), lambda i: (i, 0)),
        ],
        out_shape=[
            jax.ShapeDtypeStruct((_N, _HH), jnp.float32),
            jax.ShapeDtypeStruct((_N, _HH), jnp.float32),
        ],
    )(ha, hb, aa, ab, wa, wb, b_nn)


def _readout_body(ha_ref, hb_ref, bt_ref, w1a_ref, w1b_ref, b1_ref,
                  w2_ref, b2_ref, o_ref):
    i = pl.program_id(0)

    @pl.when(i == 0)
    def _():
        o_ref[...] = jnp.zeros_like(o_ref)

    t = (jnp.dot(ha_ref[...], w1a_ref[...],
                 preferred_element_type=jnp.float32)
         + jnp.dot(hb_ref[...], w1b_ref[...],
                   preferred_element_type=jnp.float32)
         + b1_ref[...])
    t = jnp.maximum(t, 0.0)
    y = jnp.dot(t, w2_ref[...],
                preferred_element_type=jnp.float32) + b2_ref[...]
    bt = bt_ref[0, 0, :]
    onehot = (bt[:, None] ==
              lax.broadcasted_iota(jnp.int32, (1, _NG), 1)).astype(jnp.float32)
    o_ref[...] += lax.dot_general(onehot, y, (((0,), (0,)), ((), ())),
                                  precision=_PREC,
                                  preferred_element_type=jnp.float32)


def _readout(ha, hb, batch, W1, b1, W2, b2):
    w1a = W1[:_HH]
    w1b = W1[_HH:]
    bt3 = batch.reshape(_N // _BNR, 1, _BNR)
    return pl.pallas_call(
        _readout_body,
        grid=(_N // _BNR,),
        in_specs=[
            pl.BlockSpec((_BNR, _HH), lambda i: (i, 0)),
            pl.BlockSpec((_BNR, _HH), lambda i: (i, 0)),
            pl.BlockSpec((1, 1, _BNR), lambda i: (i, 0, 0)),
            pl.BlockSpec((_HH, 1024), lambda i: (0, 0)),
            pl.BlockSpec((_HH, 1024), lambda i: (0, 0)),
            pl.BlockSpec((1024,), lambda i: (0,)),
            pl.BlockSpec((1024, 128), lambda i: (0, 0)),
            pl.BlockSpec((128,), lambda i: (0,)),
        ],
        out_specs=pl.BlockSpec((_NG, 128), lambda i: (0, 0)),
        out_shape=jax.ShapeDtypeStruct((_NG, 128), jnp.float32),
    )(ha, hb, bt3, w1a, w1b, b1, W2, b2)


def _sc_aggr(ha, hb, e2, src_p, dst2, zrows):
    mesh = plsc.VectorSubcoreMesh(core_axis_name="c", subcore_axis_name="s")

    def body(ha_hbm, hb_hbm, e_hbm, src_hbm, dst_hbm, z_hbm,
             outa_hbm, outb_hbm,
             sv0, dv0, ev0, sv1, dv1, ev1, sv2, dv2, ev2,
             lsem0, lsem1, lsem2, ssem0, ssem1, ssem2,
             gsem0, gsem1, gsem2, acc):
        c = lax.axis_index("c")
        s = lax.axis_index("s")
        pltpu.sync_copy(z_hbm, acc.at[pl.ds(s * _RPT, _RPT)])
        plsc.subcore_barrier()
        ebase0 = s * _EPT
        slots = ((sv0, dv0, ev0, lsem0, ssem0, gsem0),
                 (sv1, dv1, ev1, lsem1, ssem1, gsem1),
                 (sv2, dv2, ev2, lsem2, ssem2, gsem2))

        def slices(i):
            base = ebase0 + i * _B
            rbase = s * (_EPT // 128) + i * (_B // 128)
            return (src_hbm.at[pl.ds(base, _B)],
                    dst_hbm.at[pl.ds(rbase, _B // 128)],
                    e_hbm.at[pl.ds(c * _EPAD + base, _B)])

        def issue(i, slot):
            sv, dv, evr, lsem = slot[:4]
            ss, ds_, es = slices(i)
            pltpu.async_copy(ss, sv, lsem)
            pltpu.async_copy(ds_, dv, lsem)
            pltpu.async_copy(es, evr, lsem)

        def drain_scatter(slot):
            _, dv, evr, _, ssem, _ = slot
            for j in range(_B // 128):
                pltpu.make_async_copy(evr.at[pl.ds(j * 128, 128)],
                                      acc.at[dv.at[j]], ssem).wait()

        def load_wait_gather(i, slot):
            sv, dv, evr, lsem, _, gsem = slot
            ss, ds_, es = slices(i)
            pltpu.make_async_copy(ss, sv, lsem).wait()
            pltpu.make_async_copy(ds_, dv, lsem).wait()
            pltpu.make_async_copy(es, evr, lsem).wait()

            @pl.when(c == 0)
            def _():
                for j in range(_B // 128):
                    pltpu.async_copy(ha_hbm.at[sv.at[pl.ds(j * 128, 128)]],
                                     evr.at[pl.ds(j * 128, 128)], gsem,
                                     add=True)

            @pl.when(c == 1)
            def _():
                for j in range(_B // 128):
                    pltpu.async_copy(hb_hbm.at[sv.at[pl.ds(j * 128, 128)]],
                                     evr.at[pl.ds(j * 128, 128)], gsem,
                                     add=True)

        def phase_b(i, slot):
            sv, dv, evr, _, ssem, gsem = slot
            for j in range(_B // 128):
                pltpu.make_async_copy(ha_hbm.at[sv.at[pl.ds(j * 128, 128)]],
                                      evr.at[pl.ds(j * 128, 128)],
                                      gsem).wait()

            def relu_row(r, c2):
                evr[r, pl.ds(0, 16)] = jnp.maximum(evr[r, pl.ds(0, 16)], 0.0)
                evr[r, pl.ds(16, 16)] = jnp.maximum(evr[r, pl.ds(16, 16)], 0.0)
                return c2

            lax.fori_loop(0, _B, relu_row, 0, unroll=4)
            for j in range(_B // 128):
                pltpu.async_copy(evr.at[pl.ds(j * 128, 128)],
                                 acc.at[dv.at[j]], ssem, add=True)

        issue(0, slots[0])
        issue(1, slots[1])
        load_wait_gather(0, slots[0])

        def triple(g, carry):
            i0 = g * 3
            for q in range(3):
                i = i0 + q
                load_wait_gather(i + 1, slots[(q + 1) % 3])
                if q == 0:
                    @pl.when(g > 0)
                    def _():
                        drain_scatter(slots[2])
                else:
                    drain_scatter(slots[q - 1])
                issue(i + 2, slots[(q + 2) % 3])
                phase_b(i, slots[q])
            return carry

        lax.fori_loop(0, _NB // 3, triple, 0)
        load_wait_gather(_NB - 1, slots[1])
        drain_scatter(slots[2])
        phase_b(_NB - 2, slots[0])
        drain_scatter(slots[0])
        phase_b(_NB - 1, slots[1])
        drain_scatter(slots[1])
        plsc.subcore_barrier()

        @pl.when(c == 0)
        def _():
            @pl.when(s < 15)
            def _():
                pltpu.sync_copy(acc.at[pl.ds(s * _RPT, _RPT)],
                                outa_hbm.at[pl.ds(s * _RPT, _RPT)])

            @pl.when(s == 15)
            def _():
                pltpu.sync_copy(acc.at[pl.ds(15 * _RPT, _N - 15 * _RPT)],
                                outa_hbm.at[pl.ds(15 * _RPT, _N - 15 * _RPT)])

        @pl.when(c == 1)
        def _():
            @pl.when(s < 15)
            def _():
                pltpu.sync_copy(acc.at[pl.ds(s * _RPT, _RPT)],
                                outb_hbm.at[pl.ds(s * _RPT, _RPT)])

            @pl.when(s == 15)
            def _():
                pltpu.sync_copy(acc.at[pl.ds(15 * _RPT, _N - 15 * _RPT)],
                                outb_hbm.at[pl.ds(15 * _RPT, _N - 15 * _RPT)])

    f = pl.kernel(
        body,
        out_type=[jax.ShapeDtypeStruct((_N, _HH), jnp.float32),
                  jax.ShapeDtypeStruct((_N, _HH), jnp.float32)],
        mesh=mesh,
        compiler_params=pltpu.CompilerParams(use_tc_tiling_on_sc=False),
        scratch_types=(
            [pltpu.VMEM((_B,), jnp.int32),
             pltpu.VMEM((_B // 128, 128), jnp.int32),
             pltpu.VMEM((_B, _HH), jnp.float32)] * 3
            + [pltpu.SemaphoreType.DMA] * 9
            + [pltpu.VMEM_SHARED((_NP, _HH), jnp.float32)]
        ),
    )
    return f(ha, hb, e2.reshape(2 * _EPAD, _HH), src_p, dst2, zrows)


def kernel(x, pre_x, edge_index, edge_attr, batch,
           W_node, b_node, W_nn, b_nn,
           W_edge0, b_edge0, W_edge1, b_edge1, W_edge2, b_edge2,
           W1, b1, W2, b2):
    npad = _EPAD - _E
    src_p = jnp.concatenate([edge_index[0], jnp.zeros((npad,), jnp.int32)])
    dst_p = jnp.concatenate([edge_index[1],
                             jnp.arange(npad, dtype=jnp.int32)])
    dst2 = dst_p.reshape(_EPAD // 128, 128)
    zrows = jnp.zeros((_RPT, _HH), jnp.float32)

    ha, hb = _node_embed(x, pre_x, W_node, b_node)
    edge_lins = ((W_edge0, b_edge0), (W_edge1, b_edge1), (W_edge2, b_edge2))
    attr4 = edge_attr.reshape(_E // 4, 64)
    e2s = [_edge_proj(attr4, We, be) for We, be in edge_lins]
    for e2 in e2s:
        aa, ab = _sc_aggr(ha, hb, e2, src_p, dst2, zrows)
        ha, hb = _layer_mlp(ha, hb, aa, ab, W_nn, b_nn)
    return _readout(ha, hb, batch, W1, b1, W2, b2)
